# Initial kernel scaffold; baseline (speedup 1.0000x reference)
import jax
import jax.numpy as jnp
from jax.experimental import pallas as pl


def kernel(x, edge_index, edge_attr, batch, gn_weight, gn_bias, gn_mean_scale, We, be, W1, b1, W2, b2):
    raise NotImplementedError("write your pallas kernel here")


# speedup vs baseline: 1.3439x; 1.3439x over previous
import functools

import jax
import jax.numpy as jnp
from jax import lax
from jax.experimental import pallas as pl
from jax.experimental.pallas import tpu as pltpu
from jax.experimental.pallas import tpu_sc as plsc

N = 10000
E = 160000
D = 256
DE = 16
G = 64
H = D // 2
NS = 16
EPT = E // NS
K = 80
NBLK = EPT // K
ZR = 200
NZC = N // ZR

_PREC = lax.Precision.HIGHEST


def _dotT(a, b):
    return lax.dot_general(a, b, (((0,), (0,)), ((), ())),
                           precision=_PREC, preferred_element_type=jnp.float32)


def _dot(a, b):
    return jnp.dot(a, b, precision=_PREC, preferred_element_type=jnp.float32)


GB = 2000


def _gn_stats_body(x_ref, bat_ref, ones_ref, sum_ref, ssq_ref, cnt_ref):
    i = pl.program_id(0)
    x = x_ref[...]
    bat = bat_ref[...]
    gid = lax.broadcasted_iota(jnp.int32, (1, G), 1)
    p = (bat == gid).astype(jnp.float32)

    @pl.when(i == 0)
    def _():
        sum_ref[...] = jnp.zeros_like(sum_ref)
        ssq_ref[...] = jnp.zeros_like(ssq_ref)
        cnt_ref[...] = jnp.zeros_like(cnt_ref)

    sum_ref[...] += _dotT(p, x)
    ssq_ref[...] += _dotT(p, x * x)
    cnt_ref[...] += _dotT(p, ones_ref[...])


def _gn_norm_body(x_ref, bat_ref, w_ref, b_ref, ms_ref,
                  sum_ref, ssq_ref, cnt_ref, h0_ref, h1_ref):
    bat = bat_ref[...]
    gid = lax.broadcasted_iota(jnp.int32, (1, G), 1)
    p = (bat == gid).astype(jnp.float32)
    cnt = jnp.maximum(cnt_ref[...], 1.0)
    mean = sum_ref[...] / cnt
    ms = ms_ref[...]
    var = ssq_ref[...] / cnt + mean * mean * ms * (ms - 2.0)
    rstd = lax.rsqrt(var + 1e-5)
    out = x_ref[...] - _dot(p, mean) * ms
    h = w_ref[...] * out * _dot(p, rstd) + b_ref[...]
    h0_ref[...] = h[:, :H]
    h1_ref[...] = h[:, H:]


def _graph_norm(x, bat, w, b, ms, ones):
    xspec = pl.BlockSpec((GB, D), lambda i: (i, 0))
    bspec = pl.BlockSpec((GB, 1), lambda i: (i, 0))
    gd = pl.BlockSpec((G, D), lambda i: (0, 0))
    g1 = pl.BlockSpec((G, 1), lambda i: (0, 0))
    row = pl.BlockSpec((1, D), lambda i: (0, 0))
    sums, ssq, cnt = pl.pallas_call(
        _gn_stats_body,
        grid=(N // GB,),
        in_specs=[xspec, bspec, pl.BlockSpec((GB, 1), lambda i: (i, 0))],
        out_specs=[gd, gd, g1],
        out_shape=[jax.ShapeDtypeStruct((G, D), jnp.float32),
                   jax.ShapeDtypeStruct((G, D), jnp.float32),
                   jax.ShapeDtypeStruct((G, 1), jnp.float32)],
    )(x, bat, ones)
    return pl.pallas_call(
        _gn_norm_body,
        grid=(N // GB,),
        in_specs=[xspec, bspec, row, row, row, gd, gd, g1],
        out_specs=[pl.BlockSpec((GB, H), lambda i: (i, 0))] * 2,
        out_shape=[jax.ShapeDtypeStruct((N, H), jnp.float32)] * 2,
    )(x, bat, w, b, ms, sums, ssq, cnt)


EB = 8000


def _ee_body(ea_ref, we_ref, be_ref, e0_ref, e1_ref):
    e = _dot(ea_ref[...], we_ref[...]) + be_ref[...]
    e0_ref[...] = e[:, :H]
    e1_ref[...] = e[:, H:]


def _edge_encode(edge_attr, we, be):
    return pl.pallas_call(
        _ee_body,
        grid=(E // EB,),
        in_specs=[
            pl.BlockSpec((EB, DE), lambda i: (i, 0)),
            pl.BlockSpec((DE, D), lambda i: (0, 0)),
            pl.BlockSpec((1, D), lambda i: (0, 0)),
        ],
        out_specs=[
            pl.BlockSpec((EB, H), lambda i: (i, 0)),
            pl.BlockSpec((EB, H), lambda i: (i, 0)),
        ],
        out_shape=[jax.ShapeDtypeStruct((E, H), jnp.float32)] * 2,
    )(edge_attr, we, be)


def _sc_body(zeros_hbm, h0, h1, e0, e1, src_hbm, dst_hbm, out0, out1,
             srcb, dstb, rows, msg, agg_sh, sem):
    c = lax.axis_index("c")
    s = lax.axis_index("s")

    @pl.loop(s, NZC, step=NS)
    def _zero(i):
        pltpu.sync_copy(zeros_hbm, agg_sh.at[pl.ds(i * ZR, ZR)])

    plsc.subcore_barrier()

    def edge_loop(h_ref, e_ref):
        @pl.loop(0, NBLK)
        def _blk(blk):
            base = s * EPT + blk * K
            pltpu.sync_copy(src_hbm.at[pl.ds(base, K)], srcb)
            pltpu.sync_copy(dst_hbm.at[pl.ds(base, K)], dstb)
            pltpu.sync_copy(e_ref.at[pl.ds(base, K)], msg)
            pltpu.async_copy(h_ref.at[srcb], rows, sem).wait()

            @pl.loop(0, (K * H) // 16, unroll=8)
            def _cmp(t):
                i = t // (H // 16)
                j = (t % (H // 16)) * 16
                v = rows[i, pl.ds(j, 16)] + msg[i, pl.ds(j, 16)]
                msg[i, pl.ds(j, 16)] = jnp.maximum(v, 0.0)

            pltpu.sync_copy(msg, agg_sh.at[dstb], add=True)

    @pl.when(c == 0)
    def _():
        edge_loop(h0, e0)

    @pl.when(c == 1)
    def _():
        edge_loop(h1, e1)

    plsc.subcore_barrier()

    def writeback(out_ref):
        @pl.loop(s, NZC, step=NS)
        def _wb(i):
            sl = pl.ds(i * ZR, ZR)
            pltpu.sync_copy(agg_sh.at[sl], out_ref.at[sl])

    @pl.when(c == 0)
    def _():
        writeback(out0)

    @pl.when(c == 1)
    def _():
        writeback(out1)


@functools.cache
def _sc_edge_pass():
    return pl.kernel(
        _sc_body,
        out_type=[jax.ShapeDtypeStruct((N, H), jnp.float32)] * 2,
        mesh=plsc.VectorSubcoreMesh(core_axis_name="c", subcore_axis_name="s",
                                    num_cores=2, num_subcores=NS),
        scratch_types=[
            pltpu.VMEM((K,), jnp.int32),
            pltpu.VMEM((K,), jnp.int32),
            pltpu.VMEM((K, H), jnp.float32),
            pltpu.VMEM((K, H), jnp.float32),
            pltpu.VMEM_SHARED((N, H), jnp.float32),
            pltpu.SemaphoreType.DMA,
        ],
    )


NB = 2000


def _mlp_body(x_ref, h0_ref, h1_ref, a0_ref, a1_ref,
              w1_ref, b1_ref, w2_ref, b2_ref, o_ref):
    z0 = h0_ref[...] + a0_ref[...]
    z1 = h1_ref[...] + a1_ref[...]
    t = _dot(z0, w1_ref[:H, :]) + _dot(z1, w1_ref[H:, :]) + b1_ref[...]
    a = jnp.maximum(t, 0.0)
    o_ref[...] = x_ref[...] + _dot(a, w2_ref[...]) + b2_ref[...]


def _mlp(x, h0, h1, a0, a1, w1, b1, w2, b2):
    half = pl.BlockSpec((NB, H), lambda i: (i, 0))
    full = pl.BlockSpec((NB, D), lambda i: (i, 0))
    wspec = pl.BlockSpec((D, D), lambda i: (0, 0))
    bspec = pl.BlockSpec((1, D), lambda i: (0, 0))
    return pl.pallas_call(
        _mlp_body,
        grid=(N // NB,),
        in_specs=[full, half, half, half, half, wspec, bspec, wspec, bspec],
        out_specs=full,
        out_shape=jax.ShapeDtypeStruct((N, D), jnp.float32),
    )(x, h0, h1, a0, a1, w1, b1, w2, b2)


def kernel(x, edge_index, edge_attr, batch, gn_weight, gn_bias,
           gn_mean_scale, We, be, W1, b1, W2, b2):
    src = edge_index[0]
    dst = edge_index[1]
    bat = batch.reshape(N, 1)
    ones = jnp.ones((N, 1), jnp.float32)
    h0, h1 = _graph_norm(x, bat, gn_weight.reshape(1, D),
                         gn_bias.reshape(1, D), gn_mean_scale.reshape(1, D),
                         ones)
    e0, e1 = _edge_encode(edge_attr, We, be.reshape(1, D))
    zeros = jnp.zeros((ZR, H), jnp.float32)
    a0, a1 = _sc_edge_pass()(zeros, h0, h1, e0, e1, src, dst)
    return _mlp(x, h0, h1, a0, a1, W1, b1.reshape(1, D), W2, b2.reshape(1, D))


# speedup vs baseline: 1.7402x; 1.2949x over previous
import functools

import jax
import jax.numpy as jnp
from jax import lax
from jax.experimental import pallas as pl
from jax.experimental.pallas import tpu as pltpu
from jax.experimental.pallas import tpu_sc as plsc

N = 10000
E = 160000
D = 256
DE = 16
G = 64
H = D // 2
NS = 16
EPT = E // NS
K = 40
NBLK = EPT // K
CB = 50
NCHUNK = NBLK // CB
ZR = 200
NZC = N // ZR

_PREC = lax.Precision.HIGHEST


def _dotT(a, b):
    return lax.dot_general(a, b, (((0,), (0,)), ((), ())),
                           precision=_PREC, preferred_element_type=jnp.float32)


def _dot(a, b):
    return jnp.dot(a, b, precision=_PREC, preferred_element_type=jnp.float32)


GB = 2000


def _gn_stats_body(x_ref, bat_ref, ones_ref, sum_ref, ssq_ref, cnt_ref):
    i = pl.program_id(0)
    x = x_ref[...]
    bat = bat_ref[...]
    gid = lax.broadcasted_iota(jnp.int32, (1, G), 1)
    p = (bat == gid).astype(jnp.float32)

    @pl.when(i == 0)
    def _():
        sum_ref[...] = jnp.zeros_like(sum_ref)
        ssq_ref[...] = jnp.zeros_like(ssq_ref)
        cnt_ref[...] = jnp.zeros_like(cnt_ref)

    sum_ref[...] += _dotT(p, x)
    ssq_ref[...] += _dotT(p, x * x)
    cnt_ref[...] += _dotT(p, ones_ref[...])


def _gn_norm_body(x_ref, bat_ref, w_ref, b_ref, ms_ref,
                  sum_ref, ssq_ref, cnt_ref, h0_ref, h1_ref):
    bat = bat_ref[...]
    gid = lax.broadcasted_iota(jnp.int32, (1, G), 1)
    p = (bat == gid).astype(jnp.float32)
    cnt = jnp.maximum(cnt_ref[...], 1.0)
    mean = sum_ref[...] / cnt
    ms = ms_ref[...]
    var = ssq_ref[...] / cnt + mean * mean * ms * (ms - 2.0)
    rstd = lax.rsqrt(var + 1e-5)
    out = x_ref[...] - _dot(p, mean) * ms
    h = w_ref[...] * out * _dot(p, rstd) + b_ref[...]
    h0_ref[...] = h[:, :H]
    h1_ref[...] = h[:, H:]


def _graph_norm(x, bat, w, b, ms, ones):
    xspec = pl.BlockSpec((GB, D), lambda i: (i, 0))
    bspec = pl.BlockSpec((GB, 1), lambda i: (i, 0))
    gd = pl.BlockSpec((G, D), lambda i: (0, 0))
    g1 = pl.BlockSpec((G, 1), lambda i: (0, 0))
    row = pl.BlockSpec((1, D), lambda i: (0, 0))
    sums, ssq, cnt = pl.pallas_call(
        _gn_stats_body,
        grid=(N // GB,),
        in_specs=[xspec, bspec, pl.BlockSpec((GB, 1), lambda i: (i, 0))],
        out_specs=[gd, gd, g1],
        out_shape=[jax.ShapeDtypeStruct((G, D), jnp.float32),
                   jax.ShapeDtypeStruct((G, D), jnp.float32),
                   jax.ShapeDtypeStruct((G, 1), jnp.float32)],
    )(x, bat, ones)
    return pl.pallas_call(
        _gn_norm_body,
        grid=(N // GB,),
        in_specs=[xspec, bspec, row, row, row, gd, gd, g1],
        out_specs=[pl.BlockSpec((GB, H), lambda i: (i, 0))] * 2,
        out_shape=[jax.ShapeDtypeStruct((N, H), jnp.float32)] * 2,
    )(x, bat, w, b, ms, sums, ssq, cnt)


EB = 8000


def _ee_body(ea_ref, we_ref, be_ref, e0_ref, e1_ref):
    e = _dot(ea_ref[...], we_ref[...]) + be_ref[...]
    e0_ref[...] = e[:, :H]
    e1_ref[...] = e[:, H:]


def _edge_encode(edge_attr, we, be):
    return pl.pallas_call(
        _ee_body,
        grid=(E // EB,),
        in_specs=[
            pl.BlockSpec((EB, DE), lambda i: (i, 0)),
            pl.BlockSpec((DE, D), lambda i: (0, 0)),
            pl.BlockSpec((1, D), lambda i: (0, 0)),
        ],
        out_specs=[
            pl.BlockSpec((EB, H), lambda i: (i, 0)),
            pl.BlockSpec((EB, H), lambda i: (i, 0)),
        ],
        out_shape=[jax.ShapeDtypeStruct((E, H), jnp.float32)] * 2,
    )(edge_attr, we, be)


def _sc_body(zeros_hbm, h0, h1, e0, e1, src_hbm, dst_hbm, out0, out1,
             srcb, dstb, rows_a, rows_b, eb_a, eb_b, agg_sh,
             gs_a, gs_b, es_a, es_b, ss0, ss1):
    c = lax.axis_index("c")
    s = lax.axis_index("s")

    @pl.loop(s, NZC, step=NS)
    def _zero(i):
        pltpu.sync_copy(zeros_hbm, agg_sh.at[pl.ds(i * ZR, ZR)])

    plsc.subcore_barrier()

    def edge_loop(h_ref, e_ref):
        def gissue(cb, rbuf, sem):
            pltpu.async_copy(h_ref.at[srcb.at[cb]], rbuf, sem)

        def gwait(cb, rbuf, sem):
            pltpu.make_async_copy(h_ref.at[srcb.at[cb]], rbuf, sem).wait()

        def eissue(base, ebuf, sem):
            pltpu.async_copy(e_ref.at[pl.ds(base, K)], ebuf, sem)

        def ewait(base, ebuf, sem):
            pltpu.make_async_copy(e_ref.at[pl.ds(base, K)], ebuf, sem).wait()

        def sissue(cb, mbuf, sem):
            pltpu.async_copy(mbuf, agg_sh.at[dstb.at[cb]], sem, add=True)

        def swait(cb, mbuf, sem):
            pltpu.make_async_copy(mbuf, agg_sh.at[dstb.at[cb]], sem).wait()

        def compute(rbuf, ebuf):
            @pl.loop(0, (K * H) // 16, unroll=8)
            def _cmp(t):
                i = t // (H // 16)
                j = (t % (H // 16)) * 16
                v = rbuf[i, pl.ds(j, 16)] + ebuf[i, pl.ds(j, 16)]
                ebuf[i, pl.ds(j, 16)] = jnp.maximum(v, 0.0)

        @pl.loop(0, NCHUNK)
        def _chunk(ci):
            pltpu.sync_copy(src_hbm.at[s, ci], srcb)
            pltpu.sync_copy(dst_hbm.at[s, ci], dstb)
            base0 = s * EPT + ci * (CB * K)

            gissue(0, rows_a, gs_a)
            eissue(base0, eb_a, es_a)
            gissue(1, rows_b, gs_b)
            eissue(base0 + K, eb_b, es_b)

            @pl.loop(0, CB // 2)
            def _pair(p):
                b0 = 2 * p
                gwait(b0, rows_a, gs_a)
                ewait(base0 + b0 * K, eb_a, es_a)
                compute(rows_a, eb_a)
                sissue(b0, eb_a, ss0)
                gwait(b0 + 1, rows_b, gs_b)
                ewait(base0 + (b0 + 1) * K, eb_b, es_b)
                compute(rows_b, eb_b)
                sissue(b0 + 1, eb_b, ss1)
                swait(b0, eb_a, ss0)

                @pl.when(p < CB // 2 - 1)
                def _():
                    gissue(b0 + 2, rows_a, gs_a)
                    eissue(base0 + (b0 + 2) * K, eb_a, es_a)

                swait(b0 + 1, eb_b, ss1)

                @pl.when(p < CB // 2 - 1)
                def _():
                    gissue(b0 + 3, rows_b, gs_b)
                    eissue(base0 + (b0 + 3) * K, eb_b, es_b)

    @pl.when(c == 0)
    def _():
        edge_loop(h0, e0)

    @pl.when(c == 1)
    def _():
        edge_loop(h1, e1)

    plsc.subcore_barrier()

    def writeback(out_ref):
        @pl.loop(s, NZC, step=NS)
        def _wb(i):
            sl = pl.ds(i * ZR, ZR)
            pltpu.sync_copy(agg_sh.at[sl], out_ref.at[sl])

    @pl.when(c == 0)
    def _():
        writeback(out0)

    @pl.when(c == 1)
    def _():
        writeback(out1)


@functools.cache
def _sc_edge_pass():
    return pl.kernel(
        _sc_body,
        out_type=[jax.ShapeDtypeStruct((N, H), jnp.float32)] * 2,
        mesh=plsc.VectorSubcoreMesh(core_axis_name="c", subcore_axis_name="s",
                                    num_cores=2, num_subcores=NS),
        scratch_types=[
            pltpu.VMEM((CB, K), jnp.int32),
            pltpu.VMEM((CB, K), jnp.int32),
            pltpu.VMEM((K, H), jnp.float32),
            pltpu.VMEM((K, H), jnp.float32),
            pltpu.VMEM((K, H), jnp.float32),
            pltpu.VMEM((K, H), jnp.float32),
            pltpu.VMEM_SHARED((N, H), jnp.float32),
            pltpu.SemaphoreType.DMA,
            pltpu.SemaphoreType.DMA,
            pltpu.SemaphoreType.DMA,
            pltpu.SemaphoreType.DMA,
            pltpu.SemaphoreType.DMA,
            pltpu.SemaphoreType.DMA,
        ],
    )


NB = 2000


def _mlp_body(x_ref, h0_ref, h1_ref, a0_ref, a1_ref,
              w1_ref, b1_ref, w2_ref, b2_ref, o_ref):
    z0 = h0_ref[...] + a0_ref[...]
    z1 = h1_ref[...] + a1_ref[...]
    t = _dot(z0, w1_ref[:H, :]) + _dot(z1, w1_ref[H:, :]) + b1_ref[...]
    a = jnp.maximum(t, 0.0)
    o_ref[...] = x_ref[...] + _dot(a, w2_ref[...]) + b2_ref[...]


def _mlp(x, h0, h1, a0, a1, w1, b1, w2, b2):
    half = pl.BlockSpec((NB, H), lambda i: (i, 0))
    full = pl.BlockSpec((NB, D), lambda i: (i, 0))
    wspec = pl.BlockSpec((D, D), lambda i: (0, 0))
    bspec = pl.BlockSpec((1, D), lambda i: (0, 0))
    return pl.pallas_call(
        _mlp_body,
        grid=(N // NB,),
        in_specs=[full, half, half, half, half, wspec, bspec, wspec, bspec],
        out_specs=full,
        out_shape=jax.ShapeDtypeStruct((N, D), jnp.float32),
    )(x, h0, h1, a0, a1, w1, b1, w2, b2)


def kernel(x, edge_index, edge_attr, batch, gn_weight, gn_bias,
           gn_mean_scale, We, be, W1, b1, W2, b2):
    src = edge_index[0].reshape(NS, NCHUNK, CB, K)
    dst = edge_index[1].reshape(NS, NCHUNK, CB, K)
    bat = batch.reshape(N, 1)
    ones = jnp.ones((N, 1), jnp.float32)
    h0, h1 = _graph_norm(x, bat, gn_weight.reshape(1, D),
                         gn_bias.reshape(1, D), gn_mean_scale.reshape(1, D),
                         ones)
    e0, e1 = _edge_encode(edge_attr, We, be.reshape(1, D))
    zeros = jnp.zeros((ZR, H), jnp.float32)
    a0, a1 = _sc_edge_pass()(zeros, h0, h1, e0, e1, src, dst)
    return _mlp(x, h0, h1, a0, a1, W1, b1.reshape(1, D), W2, b2.reshape(1, D))


# speedup vs baseline: 1.9453x; 1.1179x over previous
import functools

import jax
import jax.numpy as jnp
from jax import lax
from jax.experimental import pallas as pl
from jax.experimental.pallas import tpu as pltpu
from jax.experimental.pallas import tpu_sc as plsc

N = 10000
E = 160000
D = 256
DE = 16
G = 64
H = D // 2
NS = 16
EPT = E // NS
K = 40
NBLK = EPT // K
CB = 50
NCHUNK = NBLK // CB
ZR = 200
NZC = N // ZR

_PREC = lax.Precision.HIGHEST


def _dotT(a, b):
    return lax.dot_general(a, b, (((0,), (0,)), ((), ())),
                           precision=_PREC, preferred_element_type=jnp.float32)


def _dot(a, b):
    return jnp.dot(a, b, precision=_PREC, preferred_element_type=jnp.float32)


GB = 2000


def _gn_stats_body(x_ref, bat_ref, ones_ref, sum_ref, ssq_ref, cnt_ref):
    i = pl.program_id(0)
    x = x_ref[...]
    bat = bat_ref[...]
    gid = lax.broadcasted_iota(jnp.int32, (1, G), 1)
    p = (bat == gid).astype(jnp.float32)

    @pl.when(i == 0)
    def _():
        sum_ref[...] = jnp.zeros_like(sum_ref)
        ssq_ref[...] = jnp.zeros_like(ssq_ref)
        cnt_ref[...] = jnp.zeros_like(cnt_ref)

    sum_ref[...] += _dotT(p, x)
    ssq_ref[...] += _dotT(p, x * x)
    cnt_ref[...] += _dotT(p, ones_ref[...])


def _gn_norm_body(x_ref, bat_ref, w_ref, b_ref, ms_ref,
                  sum_ref, ssq_ref, cnt_ref, h0_ref, h1_ref):
    bat = bat_ref[...]
    gid = lax.broadcasted_iota(jnp.int32, (1, G), 1)
    p = (bat == gid).astype(jnp.float32)
    cnt = jnp.maximum(cnt_ref[...], 1.0)
    mean = sum_ref[...] / cnt
    ms = ms_ref[...]
    var = ssq_ref[...] / cnt + mean * mean * ms * (ms - 2.0)
    rstd = lax.rsqrt(var + 1e-5)
    out = x_ref[...] - _dot(p, mean) * ms
    h = w_ref[...] * out * _dot(p, rstd) + b_ref[...]
    h0_ref[...] = h[:, :H]
    h1_ref[...] = h[:, H:]


def _graph_norm(x, bat, w, b, ms, ones):
    xspec = pl.BlockSpec((GB, D), lambda i: (i, 0))
    bspec = pl.BlockSpec((GB, 1), lambda i: (i, 0))
    gd = pl.BlockSpec((G, D), lambda i: (0, 0))
    g1 = pl.BlockSpec((G, 1), lambda i: (0, 0))
    row = pl.BlockSpec((1, D), lambda i: (0, 0))
    sums, ssq, cnt = pl.pallas_call(
        _gn_stats_body,
        grid=(N // GB,),
        in_specs=[xspec, bspec, pl.BlockSpec((GB, 1), lambda i: (i, 0))],
        out_specs=[gd, gd, g1],
        out_shape=[jax.ShapeDtypeStruct((G, D), jnp.float32),
                   jax.ShapeDtypeStruct((G, D), jnp.float32),
                   jax.ShapeDtypeStruct((G, 1), jnp.float32)],
    )(x, bat, ones)
    return pl.pallas_call(
        _gn_norm_body,
        grid=(N // GB,),
        in_specs=[xspec, bspec, row, row, row, gd, gd, g1],
        out_specs=[pl.BlockSpec((GB, H), lambda i: (i, 0))] * 2,
        out_shape=[jax.ShapeDtypeStruct((N, H), jnp.float32)] * 2,
    )(x, bat, w, b, ms, sums, ssq, cnt)


EB = 8000


def _ee_body(ea_ref, we_ref, be_ref, e0_ref, e1_ref):
    e = _dot(ea_ref[...], we_ref[...]) + be_ref[...]
    e0_ref[...] = e[:, :H]
    e1_ref[...] = e[:, H:]


def _edge_encode(edge_attr, we, be):
    return pl.pallas_call(
        _ee_body,
        grid=(E // EB,),
        in_specs=[
            pl.BlockSpec((EB, DE), lambda i: (i, 0)),
            pl.BlockSpec((DE, D), lambda i: (0, 0)),
            pl.BlockSpec((1, D), lambda i: (0, 0)),
        ],
        out_specs=[
            pl.BlockSpec((EB, H), lambda i: (i, 0)),
            pl.BlockSpec((EB, H), lambda i: (i, 0)),
        ],
        out_shape=[jax.ShapeDtypeStruct((E, H), jnp.float32)] * 2,
    )(edge_attr, we, be)


def _sc_body(zeros_hbm, h0, h1, e0, e1, src_hbm, dst_hbm, out0, out1,
             srcb, dstb, rows_a, rows_b, eb_a, eb_b, msg0, msg1, agg_sh,
             gs_a, gs_b, es_a, es_b, ss0, ss1):
    c = lax.axis_index("c")
    s = lax.axis_index("s")

    @pl.loop(s, NZC, step=NS)
    def _zero(i):
        pltpu.sync_copy(zeros_hbm, agg_sh.at[pl.ds(i * ZR, ZR)])

    plsc.subcore_barrier()

    def edge_loop(h_ref, e_ref):
        def gissue(cb, rbuf, sem):
            pltpu.async_copy(h_ref.at[srcb.at[cb]], rbuf, sem)

        def gwait(cb, rbuf, sem):
            pltpu.make_async_copy(h_ref.at[srcb.at[cb]], rbuf, sem).wait()

        def eissue(base, ebuf, sem):
            pltpu.async_copy(e_ref.at[pl.ds(base, K)], ebuf, sem)

        def ewait(base, ebuf, sem):
            pltpu.make_async_copy(e_ref.at[pl.ds(base, K)], ebuf, sem).wait()

        def sissue(cb, mbuf, sem):
            pltpu.async_copy(mbuf, agg_sh.at[dstb.at[cb]], sem, add=True)

        def swait(cb, mbuf, sem):
            pltpu.make_async_copy(mbuf, agg_sh.at[dstb.at[cb]], sem).wait()

        def compute(rbuf, ebuf, mbuf):
            @pl.loop(0, (K * H) // 16, unroll=8)
            def _cmp(t):
                i = t // (H // 16)
                j = (t % (H // 16)) * 16
                v = rbuf[i, pl.ds(j, 16)] + ebuf[i, pl.ds(j, 16)]
                mbuf[i, pl.ds(j, 16)] = jnp.maximum(v, 0.0)

        @pl.loop(0, NCHUNK)
        def _chunk(ci):
            pltpu.sync_copy(src_hbm.at[s, ci], srcb)
            pltpu.sync_copy(dst_hbm.at[s, ci], dstb)
            base0 = s * EPT + ci * (CB * K)

            gissue(0, rows_a, gs_a)
            eissue(base0, eb_a, es_a)
            gissue(1, rows_b, gs_b)
            eissue(base0 + K, eb_b, es_b)

            gwait(0, rows_a, gs_a)
            ewait(base0, eb_a, es_a)
            compute(rows_a, eb_a, msg0)
            sissue(0, msg0, ss0)
            gissue(2, rows_a, gs_a)
            eissue(base0 + 2 * K, eb_a, es_a)
            gwait(1, rows_b, gs_b)
            ewait(base0 + K, eb_b, es_b)
            compute(rows_b, eb_b, msg1)
            sissue(1, msg1, ss1)
            gissue(3, rows_b, gs_b)
            eissue(base0 + 3 * K, eb_b, es_b)

            @pl.loop(1, CB // 2)
            def _pair(p):
                b0 = 2 * p
                gwait(b0, rows_a, gs_a)
                ewait(base0 + b0 * K, eb_a, es_a)
                swait(b0 - 2, msg0, ss0)
                compute(rows_a, eb_a, msg0)
                sissue(b0, msg0, ss0)

                @pl.when(p < CB // 2 - 1)
                def _():
                    gissue(b0 + 2, rows_a, gs_a)
                    eissue(base0 + (b0 + 2) * K, eb_a, es_a)

                gwait(b0 + 1, rows_b, gs_b)
                ewait(base0 + (b0 + 1) * K, eb_b, es_b)
                swait(b0 - 1, msg1, ss1)
                compute(rows_b, eb_b, msg1)
                sissue(b0 + 1, msg1, ss1)

                @pl.when(p < CB // 2 - 1)
                def _():
                    gissue(b0 + 3, rows_b, gs_b)
                    eissue(base0 + (b0 + 3) * K, eb_b, es_b)

            swait(CB - 2, msg0, ss0)
            swait(CB - 1, msg1, ss1)

    @pl.when(c == 0)
    def _():
        edge_loop(h0, e0)

    @pl.when(c == 1)
    def _():
        edge_loop(h1, e1)

    plsc.subcore_barrier()

    def writeback(out_ref):
        @pl.loop(s, NZC, step=NS)
        def _wb(i):
            sl = pl.ds(i * ZR, ZR)
            pltpu.sync_copy(agg_sh.at[sl], out_ref.at[sl])

    @pl.when(c == 0)
    def _():
        writeback(out0)

    @pl.when(c == 1)
    def _():
        writeback(out1)


@functools.cache
def _sc_edge_pass():
    return pl.kernel(
        _sc_body,
        out_type=[jax.ShapeDtypeStruct((N, H), jnp.float32)] * 2,
        mesh=plsc.VectorSubcoreMesh(core_axis_name="c", subcore_axis_name="s",
                                    num_cores=2, num_subcores=NS),
        scratch_types=[
            pltpu.VMEM((CB, K), jnp.int32),
            pltpu.VMEM((CB, K), jnp.int32),
            pltpu.VMEM((K, H), jnp.float32),
            pltpu.VMEM((K, H), jnp.float32),
            pltpu.VMEM((K, H), jnp.float32),
            pltpu.VMEM((K, H), jnp.float32),
            pltpu.VMEM((K, H), jnp.float32),
            pltpu.VMEM((K, H), jnp.float32),
            pltpu.VMEM_SHARED((N, H), jnp.float32),
            pltpu.SemaphoreType.DMA,
            pltpu.SemaphoreType.DMA,
            pltpu.SemaphoreType.DMA,
            pltpu.SemaphoreType.DMA,
            pltpu.SemaphoreType.DMA,
            pltpu.SemaphoreType.DMA,
        ],
    )


NB = 2000


def _mlp_body(x_ref, h0_ref, h1_ref, a0_ref, a1_ref,
              w1_ref, b1_ref, w2_ref, b2_ref, o_ref):
    z0 = h0_ref[...] + a0_ref[...]
    z1 = h1_ref[...] + a1_ref[...]
    t = _dot(z0, w1_ref[:H, :]) + _dot(z1, w1_ref[H:, :]) + b1_ref[...]
    a = jnp.maximum(t, 0.0)
    o_ref[...] = x_ref[...] + _dot(a, w2_ref[...]) + b2_ref[...]


def _mlp(x, h0, h1, a0, a1, w1, b1, w2, b2):
    half = pl.BlockSpec((NB, H), lambda i: (i, 0))
    full = pl.BlockSpec((NB, D), lambda i: (i, 0))
    wspec = pl.BlockSpec((D, D), lambda i: (0, 0))
    bspec = pl.BlockSpec((1, D), lambda i: (0, 0))
    return pl.pallas_call(
        _mlp_body,
        grid=(N // NB,),
        in_specs=[full, half, half, half, half, wspec, bspec, wspec, bspec],
        out_specs=full,
        out_shape=jax.ShapeDtypeStruct((N, D), jnp.float32),
    )(x, h0, h1, a0, a1, w1, b1, w2, b2)


def kernel(x, edge_index, edge_attr, batch, gn_weight, gn_bias,
           gn_mean_scale, We, be, W1, b1, W2, b2):
    src = edge_index[0].reshape(NS, NCHUNK, CB, K)
    dst = edge_index[1].reshape(NS, NCHUNK, CB, K)
    bat = batch.reshape(N, 1)
    ones = jnp.ones((N, 1), jnp.float32)
    h0, h1 = _graph_norm(x, bat, gn_weight.reshape(1, D),
                         gn_bias.reshape(1, D), gn_mean_scale.reshape(1, D),
                         ones)
    e0, e1 = _edge_encode(edge_attr, We, be.reshape(1, D))
    zeros = jnp.zeros((ZR, H), jnp.float32)
    a0, a1 = _sc_edge_pass()(zeros, h0, h1, e0, e1, src, dst)
    return _mlp(x, h0, h1, a0, a1, W1, b1.reshape(1, D), W2, b2.reshape(1, D))


# speedup vs baseline: 2.0942x; 1.0765x over previous
import functools

import jax
import jax.numpy as jnp
from jax import lax
from jax.experimental import pallas as pl
from jax.experimental.pallas import tpu as pltpu
from jax.experimental.pallas import tpu_sc as plsc

N = 10000
E = 160000
D = 256
DE = 16
G = 64
H = D // 2
NS = 16
EPT = E // NS
K = 40
NBLK = EPT // K
CB = 50
NCHUNK = NBLK // CB
ZR = 200
NZC = N // ZR

_PREC = lax.Precision.HIGHEST


def _dotT(a, b):
    return lax.dot_general(a, b, (((0,), (0,)), ((), ())),
                           precision=_PREC, preferred_element_type=jnp.float32)


def _dot(a, b):
    return jnp.dot(a, b, precision=_PREC, preferred_element_type=jnp.float32)


GB = 2000


def _gn_stats_body(x_ref, bat_ref, ones_ref, sum_ref, ssq_ref, cnt_ref):
    i = pl.program_id(0)
    x = x_ref[...]
    bat = bat_ref[...]
    gid = lax.broadcasted_iota(jnp.int32, (1, G), 1)
    p = (bat == gid).astype(jnp.float32)

    @pl.when(i == 0)
    def _():
        sum_ref[...] = jnp.zeros_like(sum_ref)
        ssq_ref[...] = jnp.zeros_like(ssq_ref)
        cnt_ref[...] = jnp.zeros_like(cnt_ref)

    sum_ref[...] += _dotT(p, x)
    ssq_ref[...] += _dotT(p, x * x)
    cnt_ref[...] += _dotT(p, ones_ref[...])


def _gn_norm_body(x_ref, bat_ref, w_ref, b_ref, ms_ref,
                  sum_ref, ssq_ref, cnt_ref, h0_ref, h1_ref):
    bat = bat_ref[...]
    gid = lax.broadcasted_iota(jnp.int32, (1, G), 1)
    p = (bat == gid).astype(jnp.float32)
    cnt = jnp.maximum(cnt_ref[...], 1.0)
    mean = sum_ref[...] / cnt
    ms = ms_ref[...]
    var = ssq_ref[...] / cnt + mean * mean * ms * (ms - 2.0)
    rstd = lax.rsqrt(var + 1e-5)
    out = x_ref[...] - _dot(p, mean) * ms
    h = w_ref[...] * out * _dot(p, rstd) + b_ref[...]
    h0_ref[...] = h[:, :H]
    h1_ref[...] = h[:, H:]


def _graph_norm(x, bat, w, b, ms, ones):
    xspec = pl.BlockSpec((GB, D), lambda i: (i, 0))
    bspec = pl.BlockSpec((GB, 1), lambda i: (i, 0))
    gd = pl.BlockSpec((G, D), lambda i: (0, 0))
    g1 = pl.BlockSpec((G, 1), lambda i: (0, 0))
    row = pl.BlockSpec((1, D), lambda i: (0, 0))
    sums, ssq, cnt = pl.pallas_call(
        _gn_stats_body,
        grid=(N // GB,),
        in_specs=[xspec, bspec, pl.BlockSpec((GB, 1), lambda i: (i, 0))],
        out_specs=[gd, gd, g1],
        out_shape=[jax.ShapeDtypeStruct((G, D), jnp.float32),
                   jax.ShapeDtypeStruct((G, D), jnp.float32),
                   jax.ShapeDtypeStruct((G, 1), jnp.float32)],
    )(x, bat, ones)
    return pl.pallas_call(
        _gn_norm_body,
        grid=(N // GB,),
        in_specs=[xspec, bspec, row, row, row, gd, gd, g1],
        out_specs=[pl.BlockSpec((GB, H), lambda i: (i, 0))] * 2,
        out_shape=[jax.ShapeDtypeStruct((N, H), jnp.float32)] * 2,
    )(x, bat, w, b, ms, sums, ssq, cnt)


EB = 8000


def _ee_body(ea_ref, we_ref, be_ref, e0_ref, e1_ref):
    ea = ea_ref[...].astype(jnp.bfloat16)
    we = we_ref[...].astype(jnp.bfloat16)
    e = jnp.dot(ea, we, preferred_element_type=jnp.float32) + be_ref[...]
    e0_ref[...] = e[:, :H]
    e1_ref[...] = e[:, H:]


def _edge_encode(edge_attr, we, be):
    return pl.pallas_call(
        _ee_body,
        grid=(E // EB,),
        in_specs=[
            pl.BlockSpec((EB, DE), lambda i: (i, 0)),
            pl.BlockSpec((DE, D), lambda i: (0, 0)),
            pl.BlockSpec((1, D), lambda i: (0, 0)),
        ],
        out_specs=[
            pl.BlockSpec((EB, H), lambda i: (i, 0)),
            pl.BlockSpec((EB, H), lambda i: (i, 0)),
        ],
        out_shape=[jax.ShapeDtypeStruct((E, H), jnp.float32)] * 2,
    )(edge_attr, we, be)


def _sc_body(zeros_hbm, h0, h1, e0, e1, src_hbm, dst_hbm, out0, out1,
             srcb, dstb, rows_a, rows_b, eb_a, eb_b, msg0, msg1, agg_sh,
             gs_a, gs_b, es_a, es_b, ss0, ss1):
    c = lax.axis_index("c")
    s = lax.axis_index("s")

    @pl.loop(s, NZC, step=NS)
    def _zero(i):
        pltpu.sync_copy(zeros_hbm, agg_sh.at[pl.ds(i * ZR, ZR)])

    plsc.subcore_barrier()

    def edge_loop(h_ref, e_ref):
        def gissue(cb, rbuf, sem):
            pltpu.async_copy(h_ref.at[srcb.at[cb]], rbuf, sem)

        def gwait(cb, rbuf, sem):
            pltpu.make_async_copy(h_ref.at[srcb.at[cb]], rbuf, sem).wait()

        def eissue(base, ebuf, sem):
            pltpu.async_copy(e_ref.at[pl.ds(base, K)], ebuf, sem)

        def ewait(base, ebuf, sem):
            pltpu.make_async_copy(e_ref.at[pl.ds(base, K)], ebuf, sem).wait()

        def sissue(cb, mbuf, sem):
            pltpu.async_copy(mbuf, agg_sh.at[dstb.at[cb]], sem, add=True)

        def swait(cb, mbuf, sem):
            pltpu.make_async_copy(mbuf, agg_sh.at[dstb.at[cb]], sem).wait()

        def compute(rbuf, ebuf, mbuf):
            @pl.loop(0, (K * H) // 16, unroll=8)
            def _cmp(t):
                i = t // (H // 16)
                j = (t % (H // 16)) * 16
                v = rbuf[i, pl.ds(j, 16)] + ebuf[i, pl.ds(j, 16)]
                mbuf[i, pl.ds(j, 16)] = jnp.maximum(v, 0.0)

        @pl.loop(0, NCHUNK)
        def _chunk(ci):
            pltpu.sync_copy(src_hbm.at[s, ci], srcb)
            pltpu.sync_copy(dst_hbm.at[s, ci], dstb)
            base0 = s * EPT + ci * (CB * K)

            gissue(0, rows_a, gs_a)
            eissue(base0, eb_a, es_a)
            gissue(1, rows_b, gs_b)
            eissue(base0 + K, eb_b, es_b)

            gwait(0, rows_a, gs_a)
            ewait(base0, eb_a, es_a)
            compute(rows_a, eb_a, msg0)
            sissue(0, msg0, ss0)
            gissue(2, rows_a, gs_a)
            eissue(base0 + 2 * K, eb_a, es_a)
            gwait(1, rows_b, gs_b)
            ewait(base0 + K, eb_b, es_b)
            compute(rows_b, eb_b, msg1)
            sissue(1, msg1, ss1)
            gissue(3, rows_b, gs_b)
            eissue(base0 + 3 * K, eb_b, es_b)

            @pl.loop(1, CB // 2)
            def _pair(p):
                b0 = 2 * p
                gwait(b0, rows_a, gs_a)
                ewait(base0 + b0 * K, eb_a, es_a)
                swait(b0 - 2, msg0, ss0)
                compute(rows_a, eb_a, msg0)
                sissue(b0, msg0, ss0)

                @pl.when(p < CB // 2 - 1)
                def _():
                    gissue(b0 + 2, rows_a, gs_a)
                    eissue(base0 + (b0 + 2) * K, eb_a, es_a)

                gwait(b0 + 1, rows_b, gs_b)
                ewait(base0 + (b0 + 1) * K, eb_b, es_b)
                swait(b0 - 1, msg1, ss1)
                compute(rows_b, eb_b, msg1)
                sissue(b0 + 1, msg1, ss1)

                @pl.when(p < CB // 2 - 1)
                def _():
                    gissue(b0 + 3, rows_b, gs_b)
                    eissue(base0 + (b0 + 3) * K, eb_b, es_b)

            swait(CB - 2, msg0, ss0)
            swait(CB - 1, msg1, ss1)

    @pl.when(c == 0)
    def _():
        edge_loop(h0, e0)

    @pl.when(c == 1)
    def _():
        edge_loop(h1, e1)

    plsc.subcore_barrier()

    def writeback(out_ref):
        @pl.loop(s, NZC, step=NS)
        def _wb(i):
            sl = pl.ds(i * ZR, ZR)
            pltpu.sync_copy(agg_sh.at[sl], out_ref.at[sl])

    @pl.when(c == 0)
    def _():
        writeback(out0)

    @pl.when(c == 1)
    def _():
        writeback(out1)


@functools.cache
def _sc_edge_pass():
    return pl.kernel(
        _sc_body,
        out_type=[jax.ShapeDtypeStruct((N, H), jnp.float32)] * 2,
        mesh=plsc.VectorSubcoreMesh(core_axis_name="c", subcore_axis_name="s",
                                    num_cores=2, num_subcores=NS),
        scratch_types=[
            pltpu.VMEM((CB, K), jnp.int32),
            pltpu.VMEM((CB, K), jnp.int32),
            pltpu.VMEM((K, H), jnp.float32),
            pltpu.VMEM((K, H), jnp.float32),
            pltpu.VMEM((K, H), jnp.float32),
            pltpu.VMEM((K, H), jnp.float32),
            pltpu.VMEM((K, H), jnp.float32),
            pltpu.VMEM((K, H), jnp.float32),
            pltpu.VMEM_SHARED((N, H), jnp.float32),
            pltpu.SemaphoreType.DMA,
            pltpu.SemaphoreType.DMA,
            pltpu.SemaphoreType.DMA,
            pltpu.SemaphoreType.DMA,
            pltpu.SemaphoreType.DMA,
            pltpu.SemaphoreType.DMA,
        ],
    )


NB = 2000


def _mlp_body(x_ref, h0_ref, h1_ref, a0_ref, a1_ref,
              w1_ref, b1_ref, w2_ref, b2_ref, o_ref):
    z0 = h0_ref[...] + a0_ref[...]
    z1 = h1_ref[...] + a1_ref[...]
    t = _dot(z0, w1_ref[:H, :]) + _dot(z1, w1_ref[H:, :]) + b1_ref[...]
    a = jnp.maximum(t, 0.0)
    o_ref[...] = x_ref[...] + _dot(a, w2_ref[...]) + b2_ref[...]


def _mlp(x, h0, h1, a0, a1, w1, b1, w2, b2):
    half = pl.BlockSpec((NB, H), lambda i: (i, 0))
    full = pl.BlockSpec((NB, D), lambda i: (i, 0))
    wspec = pl.BlockSpec((D, D), lambda i: (0, 0))
    bspec = pl.BlockSpec((1, D), lambda i: (0, 0))
    return pl.pallas_call(
        _mlp_body,
        grid=(N // NB,),
        in_specs=[full, half, half, half, half, wspec, bspec, wspec, bspec],
        out_specs=full,
        out_shape=jax.ShapeDtypeStruct((N, D), jnp.float32),
    )(x, h0, h1, a0, a1, w1, b1, w2, b2)


def kernel(x, edge_index, edge_attr, batch, gn_weight, gn_bias,
           gn_mean_scale, We, be, W1, b1, W2, b2):
    src = edge_index[0].reshape(NS, NCHUNK, CB, K)
    dst = edge_index[1].reshape(NS, NCHUNK, CB, K)
    bat = batch.reshape(N, 1)
    ones = jnp.ones((N, 1), jnp.float32)
    h0, h1 = _graph_norm(x, bat, gn_weight.reshape(1, D),
                         gn_bias.reshape(1, D), gn_mean_scale.reshape(1, D),
                         ones)
    e0, e1 = _edge_encode(edge_attr, We, be.reshape(1, D))
    zeros = jnp.zeros((ZR, H), jnp.float32)
    a0, a1 = _sc_edge_pass()(zeros, h0, h1, e0, e1, src, dst)
    return _mlp(x, h0, h1, a0, a1, W1, b1.reshape(1, D), W2, b2.reshape(1, D))


# speedup vs baseline: 2.7258x; 1.3016x over previous
import functools

import jax
import jax.numpy as jnp
from jax import lax
from jax.experimental import pallas as pl
from jax.experimental.pallas import tpu as pltpu
from jax.experimental.pallas import tpu_sc as plsc

N = 10000
E = 160000
D = 256
DE = 16
G = 64
H = D // 2
NS = 16
EPT = E // NS
K = 40
NBLK = EPT // K
CB = 50
NCHUNK = NBLK // CB
ZR = 200
NZC = N // ZR

_PREC = lax.Precision.HIGHEST


def _dotT(a, b):
    return lax.dot_general(a, b, (((0,), (0,)), ((), ())),
                           precision=_PREC, preferred_element_type=jnp.float32)


def _dot(a, b):
    return jnp.dot(a, b, precision=_PREC, preferred_element_type=jnp.float32)


GB = 2000


def _gn_stats_body(x_ref, bat_ref, ones_ref, sum_ref, ssq_ref, cnt_ref):
    i = pl.program_id(0)
    x = x_ref[...]
    bat = bat_ref[...]
    gid = lax.broadcasted_iota(jnp.int32, (1, G), 1)
    p = (bat == gid).astype(jnp.float32)

    @pl.when(i == 0)
    def _():
        sum_ref[...] = jnp.zeros_like(sum_ref)
        ssq_ref[...] = jnp.zeros_like(ssq_ref)
        cnt_ref[...] = jnp.zeros_like(cnt_ref)

    sum_ref[...] += _dotT(p, x)
    ssq_ref[...] += _dotT(p, x * x)
    cnt_ref[...] += _dotT(p, ones_ref[...])


_HIMASK = -65536


def _gn_norm_body(x_ref, bat_ref, we_ref, wo_ref, be_ref, bo_ref,
                  mse_ref, mso_ref, ms_ref, sele_b_ref, selo_b_ref,
                  sele_f_ref, selo_f_ref,
                  sum_ref, ssq_ref, cnt_ref, h0_ref, h1_ref):
    bat = bat_ref[...]
    gid = lax.broadcasted_iota(jnp.int32, (1, G), 1)
    p = (bat == gid).astype(jnp.float32)
    cnt = jnp.maximum(cnt_ref[...], 1.0)
    mean = sum_ref[...] / cnt
    ms = ms_ref[...]
    var = ssq_ref[...] / cnt + mean * mean * ms * (ms - 2.0)
    rstd = lax.rsqrt(var + 1e-5)
    xb = x_ref[...].astype(jnp.bfloat16)
    xe = jnp.dot(xb, sele_b_ref[...], preferred_element_type=jnp.float32)
    xo = jnp.dot(xb, selo_b_ref[...], preferred_element_type=jnp.float32)
    mean_e = _dot(mean, sele_f_ref[...])
    mean_o = _dot(mean, selo_f_ref[...])
    rstd_e = _dot(rstd, sele_f_ref[...])
    rstd_o = _dot(rstd, selo_f_ref[...])
    out_e = xe - _dot(p, mean_e) * mse_ref[...]
    out_o = xo - _dot(p, mean_o) * mso_ref[...]
    he = we_ref[...] * out_e * _dot(p, rstd_e) + be_ref[...]
    ho = wo_ref[...] * out_o * _dot(p, rstd_o) + bo_ref[...]
    h0_ref[...] = jnp.concatenate([he[:, :H // 2], ho[:, :H // 2]], axis=1)
    h1_ref[...] = jnp.concatenate([he[:, H // 2:], ho[:, H // 2:]], axis=1)


def _graph_norm(x, bat, w, b, ms, ones):
    xspec = pl.BlockSpec((GB, D), lambda i: (i, 0))
    bspec = pl.BlockSpec((GB, 1), lambda i: (i, 0))
    gd = pl.BlockSpec((G, D), lambda i: (0, 0))
    g1 = pl.BlockSpec((G, 1), lambda i: (0, 0))
    row = pl.BlockSpec((1, D), lambda i: (0, 0))
    hrow = pl.BlockSpec((1, H), lambda i: (0, 0))
    sel = pl.BlockSpec((D, H), lambda i: (0, 0))
    sums, ssq, cnt = pl.pallas_call(
        _gn_stats_body,
        grid=(N // GB,),
        in_specs=[xspec, bspec, pl.BlockSpec((GB, 1), lambda i: (i, 0))],
        out_specs=[gd, gd, g1],
        out_shape=[jax.ShapeDtypeStruct((G, D), jnp.float32),
                   jax.ShapeDtypeStruct((G, D), jnp.float32),
                   jax.ShapeDtypeStruct((G, 1), jnp.float32)],
    )(x, bat, ones)
    ev = jnp.arange(0, D, 2)
    od = jnp.arange(1, D, 2)
    sele_f = (jnp.arange(D)[:, None] == ev[None, :]).astype(jnp.float32)
    selo_f = (jnp.arange(D)[:, None] == od[None, :]).astype(jnp.float32)
    return pl.pallas_call(
        _gn_norm_body,
        grid=(N // GB,),
        in_specs=[xspec, bspec, hrow, hrow, hrow, hrow, hrow, hrow, row,
                  sel, sel, sel, sel, gd, gd, g1],
        out_specs=[pl.BlockSpec((GB, H), lambda i: (i, 0))] * 2,
        out_shape=[jax.ShapeDtypeStruct((N, H), jnp.float32)] * 2,
    )(x, bat, w[:, ev], w[:, od], b[:, ev], b[:, od], ms[:, ev], ms[:, od],
      ms, sele_f.astype(jnp.bfloat16), selo_f.astype(jnp.bfloat16),
      sele_f, selo_f, sums, ssq, cnt)


EB = 8000


def _pack_bf16(ve, vo):
    bits_e = lax.bitcast_convert_type(
        ve.astype(jnp.bfloat16).astype(jnp.float32), jnp.int32)
    bits_o = lax.bitcast_convert_type(
        vo.astype(jnp.bfloat16).astype(jnp.float32), jnp.int32)
    return lax.shift_right_logical(bits_e, 16) | (bits_o & _HIMASK)


def _ee_body(ea_ref, wee_ref, weo_ref, bee_ref, beo_ref, e0_ref, e1_ref):
    ea = ea_ref[...].astype(jnp.bfloat16)
    ee = jnp.dot(ea, wee_ref[...], preferred_element_type=jnp.float32)
    eo = jnp.dot(ea, weo_ref[...], preferred_element_type=jnp.float32)
    packed = _pack_bf16(ee + bee_ref[...], eo + beo_ref[...])
    e0_ref[...] = packed[:, :H // 2]
    e1_ref[...] = packed[:, H // 2:]


def _edge_encode(edge_attr, we, be):
    ev = jnp.arange(0, D, 2)
    od = jnp.arange(1, D, 2)
    wspec = pl.BlockSpec((DE, H), lambda i: (0, 0))
    bspec = pl.BlockSpec((1, H), lambda i: (0, 0))
    return pl.pallas_call(
        _ee_body,
        grid=(E // EB,),
        in_specs=[
            pl.BlockSpec((EB, DE), lambda i: (i, 0)),
            wspec, wspec, bspec, bspec,
        ],
        out_specs=[
            pl.BlockSpec((EB, H // 2), lambda i: (i, 0)),
            pl.BlockSpec((EB, H // 2), lambda i: (i, 0)),
        ],
        out_shape=[jax.ShapeDtypeStruct((E, H // 2), jnp.int32)] * 2,
    )(edge_attr, we[:, ev].astype(jnp.bfloat16), we[:, od].astype(jnp.bfloat16),
      be[:, ev], be[:, od])


def _sc_body(zeros_hbm, h0, h1, e0, e1, src_hbm, dst_hbm, out0, out1,
             srcb, dstb, rows_a, rows_b, eb_a, eb_b, msg0, msg1, agg_sh,
             gs_a, gs_b, es_a, es_b, ss0, ss1):
    c = lax.axis_index("c")
    s = lax.axis_index("s")

    @pl.loop(s, NZC, step=NS)
    def _zero(i):
        pltpu.sync_copy(zeros_hbm, agg_sh.at[pl.ds(i * ZR, ZR)])

    plsc.subcore_barrier()

    def edge_loop(h_ref, e_ref):
        def gissue(cb, rbuf, sem):
            pltpu.async_copy(h_ref.at[srcb.at[cb]], rbuf, sem)

        def gwait(cb, rbuf, sem):
            pltpu.make_async_copy(h_ref.at[srcb.at[cb]], rbuf, sem).wait()

        def eissue(base, ebuf, sem):
            pltpu.async_copy(e_ref.at[pl.ds(base, K)], ebuf, sem)

        def ewait(base, ebuf, sem):
            pltpu.make_async_copy(e_ref.at[pl.ds(base, K)], ebuf, sem).wait()

        def sissue(cb, mbuf, sem):
            pltpu.async_copy(mbuf, agg_sh.at[dstb.at[cb]], sem, add=True)

        def swait(cb, mbuf, sem):
            pltpu.make_async_copy(mbuf, agg_sh.at[dstb.at[cb]], sem).wait()

        def compute(rbuf, ebuf, mbuf):
            hi_mask = -65536

            @pl.loop(0, (K * H) // 32, unroll=8)
            def _cmp(t):
                i = t // (H // 32)
                g = t % (H // 32)
                ew = ebuf[i, pl.ds(g * 16, 16)]
                bc = lambda v: lax.bitcast_convert_type(v, jnp.float32)
                lo = rbuf[i, pl.ds(g * 16, 16)] + bc(ew << 16)
                hi = rbuf[i, pl.ds(H // 2 + g * 16, 16)] + bc(ew & hi_mask)
                mbuf[i, pl.ds(g * 16, 16)] = jnp.maximum(lo, 0.0)
                mbuf[i, pl.ds(H // 2 + g * 16, 16)] = jnp.maximum(hi, 0.0)

        @pl.loop(0, NCHUNK)
        def _chunk(ci):
            pltpu.sync_copy(src_hbm.at[s, ci], srcb)
            pltpu.sync_copy(dst_hbm.at[s, ci], dstb)
            base0 = s * EPT + ci * (CB * K)

            gissue(0, rows_a, gs_a)
            eissue(base0, eb_a, es_a)
            gissue(1, rows_b, gs_b)
            eissue(base0 + K, eb_b, es_b)

            gwait(0, rows_a, gs_a)
            ewait(base0, eb_a, es_a)
            compute(rows_a, eb_a, msg0)
            sissue(0, msg0, ss0)
            gissue(2, rows_a, gs_a)
            eissue(base0 + 2 * K, eb_a, es_a)
            gwait(1, rows_b, gs_b)
            ewait(base0 + K, eb_b, es_b)
            compute(rows_b, eb_b, msg1)
            sissue(1, msg1, ss1)
            gissue(3, rows_b, gs_b)
            eissue(base0 + 3 * K, eb_b, es_b)

            @pl.loop(1, CB // 2)
            def _pair(p):
                b0 = 2 * p
                gwait(b0, rows_a, gs_a)
                ewait(base0 + b0 * K, eb_a, es_a)
                swait(b0 - 2, msg0, ss0)
                compute(rows_a, eb_a, msg0)
                sissue(b0, msg0, ss0)

                @pl.when(p < CB // 2 - 1)
                def _():
                    gissue(b0 + 2, rows_a, gs_a)
                    eissue(base0 + (b0 + 2) * K, eb_a, es_a)

                gwait(b0 + 1, rows_b, gs_b)
                ewait(base0 + (b0 + 1) * K, eb_b, es_b)
                swait(b0 - 1, msg1, ss1)
                compute(rows_b, eb_b, msg1)
                sissue(b0 + 1, msg1, ss1)

                @pl.when(p < CB // 2 - 1)
                def _():
                    gissue(b0 + 3, rows_b, gs_b)
                    eissue(base0 + (b0 + 3) * K, eb_b, es_b)

            swait(CB - 2, msg0, ss0)
            swait(CB - 1, msg1, ss1)

    @pl.when(c == 0)
    def _():
        edge_loop(h0, e0)

    @pl.when(c == 1)
    def _():
        edge_loop(h1, e1)

    plsc.subcore_barrier()

    def writeback(out_ref):
        @pl.loop(s, NZC, step=NS)
        def _wb(i):
            sl = pl.ds(i * ZR, ZR)
            pltpu.sync_copy(agg_sh.at[sl], out_ref.at[sl])

    @pl.when(c == 0)
    def _():
        writeback(out0)

    @pl.when(c == 1)
    def _():
        writeback(out1)


@functools.cache
def _sc_edge_pass():
    return pl.kernel(
        _sc_body,
        out_type=[jax.ShapeDtypeStruct((N, H), jnp.float32)] * 2,
        mesh=plsc.VectorSubcoreMesh(core_axis_name="c", subcore_axis_name="s",
                                    num_cores=2, num_subcores=NS),
        scratch_types=[
            pltpu.VMEM((CB, K), jnp.int32),
            pltpu.VMEM((CB, K), jnp.int32),
            pltpu.VMEM((K, H), jnp.float32),
            pltpu.VMEM((K, H), jnp.float32),
            pltpu.VMEM((K, H // 2), jnp.int32),
            pltpu.VMEM((K, H // 2), jnp.int32),
            pltpu.VMEM((K, H), jnp.float32),
            pltpu.VMEM((K, H), jnp.float32),
            pltpu.VMEM_SHARED((N, H), jnp.float32),
            pltpu.SemaphoreType.DMA,
            pltpu.SemaphoreType.DMA,
            pltpu.SemaphoreType.DMA,
            pltpu.SemaphoreType.DMA,
            pltpu.SemaphoreType.DMA,
            pltpu.SemaphoreType.DMA,
        ],
    )


NB = 2000


def _mlp_body(x_ref, h0_ref, h1_ref, a0_ref, a1_ref,
              w1_ref, b1_ref, w2_ref, b2_ref, o_ref):
    z0 = h0_ref[...] + a0_ref[...]
    z1 = h1_ref[...] + a1_ref[...]
    t = _dot(z0, w1_ref[:H, :]) + _dot(z1, w1_ref[H:, :]) + b1_ref[...]
    a = jnp.maximum(t, 0.0)
    o_ref[...] = x_ref[...] + _dot(a, w2_ref[...]) + b2_ref[...]


def _mlp(x, h0, h1, a0, a1, w1, b1, w2, b2):
    half = pl.BlockSpec((NB, H), lambda i: (i, 0))
    full = pl.BlockSpec((NB, D), lambda i: (i, 0))
    wspec = pl.BlockSpec((D, D), lambda i: (0, 0))
    bspec = pl.BlockSpec((1, D), lambda i: (0, 0))
    return pl.pallas_call(
        _mlp_body,
        grid=(N // NB,),
        in_specs=[full, half, half, half, half, wspec, bspec, wspec, bspec],
        out_specs=full,
        out_shape=jax.ShapeDtypeStruct((N, D), jnp.float32),
    )(x, h0, h1, a0, a1, w1, b1, w2, b2)


def kernel(x, edge_index, edge_attr, batch, gn_weight, gn_bias,
           gn_mean_scale, We, be, W1, b1, W2, b2):
    src = edge_index[0].reshape(NS, NCHUNK, CB, K)
    dst = edge_index[1].reshape(NS, NCHUNK, CB, K)
    bat = batch.reshape(N, 1)
    ones = jnp.ones((N, 1), jnp.float32)
    h0, h1 = _graph_norm(x, bat, gn_weight.reshape(1, D),
                         gn_bias.reshape(1, D), gn_mean_scale.reshape(1, D),
                         ones)
    e0, e1 = _edge_encode(edge_attr, We, be.reshape(1, D))
    zeros = jnp.zeros((ZR, H), jnp.float32)
    a0, a1 = _sc_edge_pass()(zeros, h0, h1, e0, e1, src, dst)
    permh = jnp.concatenate([jnp.arange(0, H, 2), jnp.arange(1, H, 2)])
    w1m = W1[jnp.concatenate([permh, permh + H]), :]
    return _mlp(x, h0, h1, a0, a1, w1m, b1.reshape(1, D), W2,
                b2.reshape(1, D))


# speedup vs baseline: 2.7411x; 1.0056x over previous
import functools

import jax
import jax.numpy as jnp
from jax import lax
from jax.experimental import pallas as pl
from jax.experimental.pallas import tpu as pltpu
from jax.experimental.pallas import tpu_sc as plsc

N = 10000
E = 160000
D = 256
DE = 16
G = 64
H = D // 2
NS = 16
EPT = E // NS
K = 40
NBLK = EPT // K
CB = 50
NCHUNK = NBLK // CB
ZR = 200
NZC = N // ZR

_PREC = lax.Precision.HIGHEST


def _dotT(a, b):
    return lax.dot_general(a, b, (((0,), (0,)), ((), ())),
                           precision=_PREC, preferred_element_type=jnp.float32)


def _dot(a, b):
    return jnp.dot(a, b, precision=_PREC, preferred_element_type=jnp.float32)


GB = 2000


def _gn_stats_body(x_ref, bat_ref, ones_ref, sum_ref, ssq_ref, cnt_ref):
    i = pl.program_id(0)
    x = x_ref[...]
    bat = bat_ref[...]
    gid = lax.broadcasted_iota(jnp.int32, (1, G), 1)
    p = (bat == gid).astype(jnp.float32)

    @pl.when(i == 0)
    def _():
        sum_ref[...] = jnp.zeros_like(sum_ref)
        ssq_ref[...] = jnp.zeros_like(ssq_ref)
        cnt_ref[...] = jnp.zeros_like(cnt_ref)

    sum_ref[...] += _dotT(p, x)
    ssq_ref[...] += _dotT(p, x * x)
    cnt_ref[...] += _dotT(p, ones_ref[...])


_HIMASK = -65536


def _gn_norm_body(x_ref, bat_ref, we_ref, wo_ref, be_ref, bo_ref,
                  mse_ref, mso_ref, ms_ref, sele_b_ref, selo_b_ref,
                  sele_f_ref, selo_f_ref,
                  sum_ref, ssq_ref, cnt_ref, h0_ref, h1_ref):
    bat = bat_ref[...]
    gid = lax.broadcasted_iota(jnp.int32, (1, G), 1)
    p = (bat == gid).astype(jnp.float32)
    cnt = jnp.maximum(cnt_ref[...], 1.0)
    mean = sum_ref[...] / cnt
    ms = ms_ref[...]
    var = ssq_ref[...] / cnt + mean * mean * ms * (ms - 2.0)
    rstd = lax.rsqrt(var + 1e-5)
    xb = x_ref[...].astype(jnp.bfloat16)
    xe = jnp.dot(xb, sele_b_ref[...], preferred_element_type=jnp.float32)
    xo = jnp.dot(xb, selo_b_ref[...], preferred_element_type=jnp.float32)
    mean_e = _dot(mean, sele_f_ref[...])
    mean_o = _dot(mean, selo_f_ref[...])
    rstd_e = _dot(rstd, sele_f_ref[...])
    rstd_o = _dot(rstd, selo_f_ref[...])
    out_e = xe - _dot(p, mean_e) * mse_ref[...]
    out_o = xo - _dot(p, mean_o) * mso_ref[...]
    he = we_ref[...] * out_e * _dot(p, rstd_e) + be_ref[...]
    ho = wo_ref[...] * out_o * _dot(p, rstd_o) + bo_ref[...]
    h0_ref[...] = jnp.concatenate([he[:, :H // 2], ho[:, :H // 2]], axis=1)
    h1_ref[...] = jnp.concatenate([he[:, H // 2:], ho[:, H // 2:]], axis=1)


NSTEP_GN = N // GB
NSTEP_EE = 20
EB = E // NSTEP_EE


def _pack_bf16(ve, vo):
    bits_e = lax.bitcast_convert_type(
        ve.astype(jnp.bfloat16).astype(jnp.float32), jnp.int32)
    bits_o = lax.bitcast_convert_type(
        vo.astype(jnp.bfloat16).astype(jnp.float32), jnp.int32)
    return lax.shift_right_logical(bits_e, 16) | (bits_o & _HIMASK)


def _ee_body(ea_ref, wee_ref, weo_ref, bee_ref, beo_ref, e0_ref, e1_ref):
    ea = ea_ref[...].astype(jnp.bfloat16)
    ee = jnp.dot(ea, wee_ref[...], preferred_element_type=jnp.float32)
    eo = jnp.dot(ea, weo_ref[...], preferred_element_type=jnp.float32)
    packed = _pack_bf16(ee + bee_ref[...], eo + beo_ref[...])
    e0_ref[...] = packed[:, :H // 2]
    e1_ref[...] = packed[:, H // 2:]


def _pre_body(x_ref, bat_ref, ones_ref, we_ref, wo_ref, be_ref, bo_ref,
              mse_ref, mso_ref, ms_ref, sele_b_ref, selo_b_ref,
              sele_f_ref, selo_f_ref, ea_ref, wee_ref, weo_ref,
              bee_ref, beo_ref, h0_ref, h1_ref, e0_ref, e1_ref,
              sum_s, ssq_s, cnt_s):
    i = pl.program_id(0)

    @pl.when(i < NSTEP_GN)
    def _():
        _gn_stats_body(x_ref, bat_ref, ones_ref, sum_s, ssq_s, cnt_s)

    @pl.when((i >= NSTEP_GN) & (i < 2 * NSTEP_GN))
    def _():
        _gn_norm_body(x_ref, bat_ref, we_ref, wo_ref, be_ref, bo_ref,
                      mse_ref, mso_ref, ms_ref, sele_b_ref, selo_b_ref,
                      sele_f_ref, selo_f_ref, sum_s, ssq_s, cnt_s,
                      h0_ref, h1_ref)

    @pl.when(i >= 2 * NSTEP_GN)
    def _():
        _ee_body(ea_ref, wee_ref, weo_ref, bee_ref, beo_ref, e0_ref, e1_ref)


def _pre_sc(x, bat, w, b, ms, ones, edge_attr, we, be):
    ev = jnp.arange(0, D, 2)
    od = jnp.arange(1, D, 2)
    sele_f = (jnp.arange(D)[:, None] == ev[None, :]).astype(jnp.float32)
    selo_f = (jnp.arange(D)[:, None] == od[None, :]).astype(jnp.float32)

    def xmap(i):
        return (jnp.clip(jnp.where(i < NSTEP_GN, i, i - NSTEP_GN),
                         0, NSTEP_GN - 1), 0)

    def eamap(i):
        return (jnp.clip(i - 2 * NSTEP_GN, 0, NSTEP_EE - 1), 0)

    def hmap(i):
        return (jnp.clip(i - NSTEP_GN, 0, NSTEP_GN - 1), 0)

    xspec = pl.BlockSpec((GB, D), xmap)
    bspec = pl.BlockSpec((GB, 1), xmap)
    row = pl.BlockSpec((1, D), lambda i: (0, 0))
    hrow = pl.BlockSpec((1, H), lambda i: (0, 0))
    sel = pl.BlockSpec((D, H), lambda i: (0, 0))
    wspec = pl.BlockSpec((DE, H), lambda i: (0, 0))
    return pl.pallas_call(
        _pre_body,
        grid=(2 * NSTEP_GN + NSTEP_EE,),
        in_specs=[xspec, bspec, bspec, hrow, hrow, hrow, hrow, hrow, hrow,
                  row, sel, sel, sel, sel,
                  pl.BlockSpec((EB, DE), eamap), wspec, wspec, hrow, hrow],
        out_specs=[pl.BlockSpec((GB, H), hmap)] * 2 +
                  [pl.BlockSpec((EB, H // 2), eamap)] * 2,
        out_shape=[jax.ShapeDtypeStruct((N, H), jnp.float32)] * 2 +
                  [jax.ShapeDtypeStruct((E, H // 2), jnp.int32)] * 2,
        scratch_shapes=[pltpu.VMEM((G, D), jnp.float32),
                        pltpu.VMEM((G, D), jnp.float32),
                        pltpu.VMEM((G, 1), jnp.float32)],
    )(x, bat, ones, w[:, ev], w[:, od], b[:, ev], b[:, od], ms[:, ev],
      ms[:, od], ms, sele_f.astype(jnp.bfloat16), selo_f.astype(jnp.bfloat16),
      sele_f, selo_f, edge_attr, we[:, ev].astype(jnp.bfloat16),
      we[:, od].astype(jnp.bfloat16), be[:, ev], be[:, od])


def _sc_body(zeros_hbm, h0, h1, e0, e1, src_hbm, dst_hbm, out0, out1,
             srcb, dstb, rows_a, rows_b, eb_a, eb_b, msg0, msg1, agg_sh,
             gs_a, gs_b, es_a, es_b, ss0, ss1):
    c = lax.axis_index("c")
    s = lax.axis_index("s")

    @pl.loop(s, NZC, step=NS)
    def _zero(i):
        pltpu.sync_copy(zeros_hbm, agg_sh.at[pl.ds(i * ZR, ZR)])

    plsc.subcore_barrier()

    def edge_loop(h_ref, e_ref):
        def gissue(cb, rbuf, sem):
            pltpu.async_copy(h_ref.at[srcb.at[cb]], rbuf, sem)

        def gwait(cb, rbuf, sem):
            pltpu.make_async_copy(h_ref.at[srcb.at[cb]], rbuf, sem).wait()

        def eissue(base, ebuf, sem):
            pltpu.async_copy(e_ref.at[pl.ds(base, K)], ebuf, sem)

        def ewait(base, ebuf, sem):
            pltpu.make_async_copy(e_ref.at[pl.ds(base, K)], ebuf, sem).wait()

        def sissue(cb, mbuf, sem):
            pltpu.async_copy(mbuf, agg_sh.at[dstb.at[cb]], sem, add=True)

        def swait(cb, mbuf, sem):
            pltpu.make_async_copy(mbuf, agg_sh.at[dstb.at[cb]], sem).wait()

        def compute(rbuf, ebuf, mbuf):
            hi_mask = -65536

            @pl.loop(0, (K * H) // 32, unroll=8)
            def _cmp(t):
                i = t // (H // 32)
                g = t % (H // 32)
                ew = ebuf[i, pl.ds(g * 16, 16)]
                bc = lambda v: lax.bitcast_convert_type(v, jnp.float32)
                lo = rbuf[i, pl.ds(g * 16, 16)] + bc(ew << 16)
                hi = rbuf[i, pl.ds(H // 2 + g * 16, 16)] + bc(ew & hi_mask)
                mbuf[i, pl.ds(g * 16, 16)] = jnp.maximum(lo, 0.0)
                mbuf[i, pl.ds(H // 2 + g * 16, 16)] = jnp.maximum(hi, 0.0)

        @pl.loop(0, NCHUNK)
        def _chunk(ci):
            pltpu.sync_copy(src_hbm.at[s, ci], srcb)
            pltpu.sync_copy(dst_hbm.at[s, ci], dstb)
            base0 = s * EPT + ci * (CB * K)

            gissue(0, rows_a, gs_a)
            eissue(base0, eb_a, es_a)
            gissue(1, rows_b, gs_b)
            eissue(base0 + K, eb_b, es_b)

            gwait(0, rows_a, gs_a)
            ewait(base0, eb_a, es_a)
            compute(rows_a, eb_a, msg0)
            sissue(0, msg0, ss0)
            gissue(2, rows_a, gs_a)
            eissue(base0 + 2 * K, eb_a, es_a)
            gwait(1, rows_b, gs_b)
            ewait(base0 + K, eb_b, es_b)
            compute(rows_b, eb_b, msg1)
            sissue(1, msg1, ss1)
            gissue(3, rows_b, gs_b)
            eissue(base0 + 3 * K, eb_b, es_b)

            @pl.loop(1, CB // 2)
            def _pair(p):
                b0 = 2 * p
                gwait(b0, rows_a, gs_a)
                ewait(base0 + b0 * K, eb_a, es_a)
                swait(b0 - 2, msg0, ss0)
                compute(rows_a, eb_a, msg0)
                sissue(b0, msg0, ss0)

                @pl.when(p < CB // 2 - 1)
                def _():
                    gissue(b0 + 2, rows_a, gs_a)
                    eissue(base0 + (b0 + 2) * K, eb_a, es_a)

                gwait(b0 + 1, rows_b, gs_b)
                ewait(base0 + (b0 + 1) * K, eb_b, es_b)
                swait(b0 - 1, msg1, ss1)
                compute(rows_b, eb_b, msg1)
                sissue(b0 + 1, msg1, ss1)

                @pl.when(p < CB // 2 - 1)
                def _():
                    gissue(b0 + 3, rows_b, gs_b)
                    eissue(base0 + (b0 + 3) * K, eb_b, es_b)

            swait(CB - 2, msg0, ss0)
            swait(CB - 1, msg1, ss1)

    @pl.when(c == 0)
    def _():
        edge_loop(h0, e0)

    @pl.when(c == 1)
    def _():
        edge_loop(h1, e1)

    plsc.subcore_barrier()

    def writeback(out_ref):
        @pl.loop(s, NZC, step=NS)
        def _wb(i):
            sl = pl.ds(i * ZR, ZR)
            pltpu.sync_copy(agg_sh.at[sl], out_ref.at[sl])

    @pl.when(c == 0)
    def _():
        writeback(out0)

    @pl.when(c == 1)
    def _():
        writeback(out1)


@functools.cache
def _sc_edge_pass():
    return pl.kernel(
        _sc_body,
        out_type=[jax.ShapeDtypeStruct((N, H), jnp.float32)] * 2,
        mesh=plsc.VectorSubcoreMesh(core_axis_name="c", subcore_axis_name="s",
                                    num_cores=2, num_subcores=NS),
        scratch_types=[
            pltpu.VMEM((CB, K), jnp.int32),
            pltpu.VMEM((CB, K), jnp.int32),
            pltpu.VMEM((K, H), jnp.float32),
            pltpu.VMEM((K, H), jnp.float32),
            pltpu.VMEM((K, H // 2), jnp.int32),
            pltpu.VMEM((K, H // 2), jnp.int32),
            pltpu.VMEM((K, H), jnp.float32),
            pltpu.VMEM((K, H), jnp.float32),
            pltpu.VMEM_SHARED((N, H), jnp.float32),
            pltpu.SemaphoreType.DMA,
            pltpu.SemaphoreType.DMA,
            pltpu.SemaphoreType.DMA,
            pltpu.SemaphoreType.DMA,
            pltpu.SemaphoreType.DMA,
            pltpu.SemaphoreType.DMA,
        ],
    )


NB = 2000


def _mlp_body(x_ref, h0_ref, h1_ref, a0_ref, a1_ref,
              w1_ref, b1_ref, w2_ref, b2_ref, o_ref):
    z0 = h0_ref[...] + a0_ref[...]
    z1 = h1_ref[...] + a1_ref[...]
    t = _dot(z0, w1_ref[:H, :]) + _dot(z1, w1_ref[H:, :]) + b1_ref[...]
    a = jnp.maximum(t, 0.0)
    o_ref[...] = x_ref[...] + _dot(a, w2_ref[...]) + b2_ref[...]


def _mlp(x, h0, h1, a0, a1, w1, b1, w2, b2):
    half = pl.BlockSpec((NB, H), lambda i: (i, 0))
    full = pl.BlockSpec((NB, D), lambda i: (i, 0))
    wspec = pl.BlockSpec((D, D), lambda i: (0, 0))
    bspec = pl.BlockSpec((1, D), lambda i: (0, 0))
    return pl.pallas_call(
        _mlp_body,
        grid=(N // NB,),
        in_specs=[full, half, half, half, half, wspec, bspec, wspec, bspec],
        out_specs=full,
        out_shape=jax.ShapeDtypeStruct((N, D), jnp.float32),
    )(x, h0, h1, a0, a1, w1, b1, w2, b2)


def kernel(x, edge_index, edge_attr, batch, gn_weight, gn_bias,
           gn_mean_scale, We, be, W1, b1, W2, b2):
    src = edge_index[0].reshape(NS, NCHUNK, CB, K)
    dst = edge_index[1].reshape(NS, NCHUNK, CB, K)
    bat = batch.reshape(N, 1)
    ones = jnp.ones((N, 1), jnp.float32)
    h0, h1, e0, e1 = _pre_sc(x, bat, gn_weight.reshape(1, D),
                             gn_bias.reshape(1, D),
                             gn_mean_scale.reshape(1, D), ones,
                             edge_attr, We, be.reshape(1, D))
    zeros = jnp.zeros((ZR, H), jnp.float32)
    a0, a1 = _sc_edge_pass()(zeros, h0, h1, e0, e1, src, dst)
    permh = jnp.concatenate([jnp.arange(0, H, 2), jnp.arange(1, H, 2)])
    w1m = W1[jnp.concatenate([permh, permh + H]), :]
    return _mlp(x, h0, h1, a0, a1, w1m, b1.reshape(1, D), W2,
                b2.reshape(1, D))


# speedup vs baseline: 2.8763x; 1.0493x over previous
import functools

import jax
import jax.numpy as jnp
from jax import lax
from jax.experimental import pallas as pl
from jax.experimental.pallas import tpu as pltpu
from jax.experimental.pallas import tpu_sc as plsc

N = 10000
E = 160000
D = 256
DE = 16
G = 64
H = D // 2
NS = 16
EPT = E // NS
K = 40
NBLK = EPT // K
CB = 50
NCHUNK = NBLK // CB
ZR = 200
NZC = N // ZR

_PREC = lax.Precision.HIGHEST


def _dotT(a, b):
    return lax.dot_general(a, b, (((0,), (0,)), ((), ())),
                           precision=_PREC, preferred_element_type=jnp.float32)


def _dot(a, b):
    return jnp.dot(a, b, precision=_PREC, preferred_element_type=jnp.float32)


GB = 2000


def _gn_stats_body(x_ref, bat_ref, ones_ref, sum_ref, ssq_ref, cnt_ref):
    i = pl.program_id(0)
    x = x_ref[...]
    bat = bat_ref[...]
    gid = lax.broadcasted_iota(jnp.int32, (1, G), 1)
    p = (bat == gid).astype(jnp.float32)

    @pl.when(i == 0)
    def _():
        sum_ref[...] = jnp.zeros_like(sum_ref)
        ssq_ref[...] = jnp.zeros_like(ssq_ref)
        cnt_ref[...] = jnp.zeros_like(cnt_ref)

    sum_ref[...] += _dotT(p, x)
    ssq_ref[...] += _dotT(p, x * x)
    cnt_ref[...] += _dotT(p, ones_ref[...])


_HIMASK = -65536


def _gn_norm_body(x_ref, bat_ref, we_ref, wo_ref, be_ref, bo_ref,
                  mse_ref, mso_ref, ms_ref, sele_b_ref, selo_b_ref,
                  sele_f_ref, selo_f_ref,
                  sum_ref, ssq_ref, cnt_ref, h0_ref, h1_ref):
    bat = bat_ref[...]
    gid = lax.broadcasted_iota(jnp.int32, (1, G), 1)
    p = (bat == gid).astype(jnp.float32)
    cnt = jnp.maximum(cnt_ref[...], 1.0)
    mean = sum_ref[...] / cnt
    ms = ms_ref[...]
    var = ssq_ref[...] / cnt + mean * mean * ms * (ms - 2.0)
    rstd = lax.rsqrt(var + 1e-5)
    xb = x_ref[...].astype(jnp.bfloat16)
    xe = jnp.dot(xb, sele_b_ref[...], preferred_element_type=jnp.float32)
    xo = jnp.dot(xb, selo_b_ref[...], preferred_element_type=jnp.float32)
    mean_e = _dot(mean, sele_f_ref[...])
    mean_o = _dot(mean, selo_f_ref[...])
    rstd_e = _dot(rstd, sele_f_ref[...])
    rstd_o = _dot(rstd, selo_f_ref[...])
    out_e = xe - _dot(p, mean_e) * mse_ref[...]
    out_o = xo - _dot(p, mean_o) * mso_ref[...]
    he = we_ref[...] * out_e * _dot(p, rstd_e) + be_ref[...]
    ho = wo_ref[...] * out_o * _dot(p, rstd_o) + bo_ref[...]
    h0_ref[...] = jnp.concatenate([he[:, :H // 2], ho[:, :H // 2]], axis=1)
    h1_ref[...] = jnp.concatenate([he[:, H // 2:], ho[:, H // 2:]], axis=1)


NSTEP_GN = N // GB
NSTEP_EE = 20
EB = E // NSTEP_EE


def _pack_bf16(ve, vo):
    bits_e = lax.bitcast_convert_type(
        ve.astype(jnp.bfloat16).astype(jnp.float32), jnp.int32)
    bits_o = lax.bitcast_convert_type(
        vo.astype(jnp.bfloat16).astype(jnp.float32), jnp.int32)
    return lax.shift_right_logical(bits_e, 16) | (bits_o & _HIMASK)


def _ee_body(ea_ref, wee_ref, weo_ref, bee_ref, beo_ref, e0_ref, e1_ref):
    ea = ea_ref[...].astype(jnp.bfloat16)
    ee = jnp.dot(ea, wee_ref[...], preferred_element_type=jnp.float32)
    eo = jnp.dot(ea, weo_ref[...], preferred_element_type=jnp.float32)
    packed = _pack_bf16(ee + bee_ref[...], eo + beo_ref[...])
    e0_ref[...] = packed[:, :H // 2]
    e1_ref[...] = packed[:, H // 2:]


def _pre_body(x_ref, bat_ref, ones_ref, we_ref, wo_ref, be_ref, bo_ref,
              mse_ref, mso_ref, ms_ref, sele_b_ref, selo_b_ref,
              sele_f_ref, selo_f_ref, ea_ref, wee_ref, weo_ref,
              bee_ref, beo_ref, h0_ref, h1_ref, e0_ref, e1_ref,
              sum_s, ssq_s, cnt_s):
    i = pl.program_id(0)

    @pl.when(i < NSTEP_GN)
    def _():
        _gn_stats_body(x_ref, bat_ref, ones_ref, sum_s, ssq_s, cnt_s)

    @pl.when((i >= NSTEP_GN) & (i < 2 * NSTEP_GN))
    def _():
        _gn_norm_body(x_ref, bat_ref, we_ref, wo_ref, be_ref, bo_ref,
                      mse_ref, mso_ref, ms_ref, sele_b_ref, selo_b_ref,
                      sele_f_ref, selo_f_ref, sum_s, ssq_s, cnt_s,
                      h0_ref, h1_ref)

    @pl.when(i >= 2 * NSTEP_GN)
    def _():
        _ee_body(ea_ref, wee_ref, weo_ref, bee_ref, beo_ref, e0_ref, e1_ref)


def _pre_sc(x, bat, w, b, ms, ones, edge_attr, we, be):
    ev = jnp.arange(0, D, 2)
    od = jnp.arange(1, D, 2)
    sele_f = (jnp.arange(D)[:, None] == ev[None, :]).astype(jnp.float32)
    selo_f = (jnp.arange(D)[:, None] == od[None, :]).astype(jnp.float32)

    def xmap(i):
        return (jnp.clip(jnp.where(i < NSTEP_GN, i, i - NSTEP_GN),
                         0, NSTEP_GN - 1), 0)

    def eamap(i):
        return (jnp.clip(i - 2 * NSTEP_GN, 0, NSTEP_EE - 1), 0)

    def hmap(i):
        return (jnp.clip(i - NSTEP_GN, 0, NSTEP_GN - 1), 0)

    xspec = pl.BlockSpec((GB, D), xmap)
    bspec = pl.BlockSpec((GB, 1), xmap)
    row = pl.BlockSpec((1, D), lambda i: (0, 0))
    hrow = pl.BlockSpec((1, H), lambda i: (0, 0))
    sel = pl.BlockSpec((D, H), lambda i: (0, 0))
    wspec = pl.BlockSpec((DE, H), lambda i: (0, 0))
    return pl.pallas_call(
        _pre_body,
        grid=(2 * NSTEP_GN + NSTEP_EE,),
        in_specs=[xspec, bspec, bspec, hrow, hrow, hrow, hrow, hrow, hrow,
                  row, sel, sel, sel, sel,
                  pl.BlockSpec((EB, DE), eamap), wspec, wspec, hrow, hrow],
        out_specs=[pl.BlockSpec((GB, H), hmap)] * 2 +
                  [pl.BlockSpec((EB, H // 2), eamap)] * 2,
        out_shape=[jax.ShapeDtypeStruct((N, H), jnp.float32)] * 2 +
                  [jax.ShapeDtypeStruct((E, H // 2), jnp.int32)] * 2,
        scratch_shapes=[pltpu.VMEM((G, D), jnp.float32),
                        pltpu.VMEM((G, D), jnp.float32),
                        pltpu.VMEM((G, 1), jnp.float32)],
    )(x, bat, ones, w[:, ev], w[:, od], b[:, ev], b[:, od], ms[:, ev],
      ms[:, od], ms, sele_f.astype(jnp.bfloat16), selo_f.astype(jnp.bfloat16),
      sele_f, selo_f, edge_attr, we[:, ev].astype(jnp.bfloat16),
      we[:, od].astype(jnp.bfloat16), be[:, ev], be[:, od])


def _sc_body(zeros_hbm, h0, h1, e0, e1, src_hbm, dst_hbm, out0, out1,
             srcb, dstb, rows_a, rows_b, eb_a, eb_b, msg0, msg1, agg_sh,
             gs_a, gs_b, es_a, es_b, ss0, ss1):
    c = lax.axis_index("c")
    s = lax.axis_index("s")

    @pl.loop(s, NZC, step=NS)
    def _zero(i):
        pltpu.sync_copy(zeros_hbm, agg_sh.at[pl.ds(i * ZR, ZR)])

    plsc.subcore_barrier()

    def edge_loop(h_ref, e_ref):
        def gissue(cb, rbuf, sem):
            pltpu.async_copy(h_ref.at[srcb.at[cb]], rbuf, sem)

        def gwait(cb, rbuf, sem):
            pltpu.make_async_copy(h_ref.at[srcb.at[cb]], rbuf, sem).wait()

        def eissue(base, ebuf, sem):
            pltpu.async_copy(e_ref.at[pl.ds(base, K)], ebuf, sem)

        def ewait(base, ebuf, sem):
            pltpu.make_async_copy(e_ref.at[pl.ds(base, K)], ebuf, sem).wait()

        def sissue(cb, mbuf, sem):
            pltpu.async_copy(mbuf, agg_sh.at[dstb.at[cb]], sem, add=True)

        def swait(cb, mbuf, sem):
            pltpu.make_async_copy(mbuf, agg_sh.at[dstb.at[cb]], sem).wait()

        def compute(rbuf, ebuf, mbuf):
            hi_mask = -65536

            @pl.loop(0, (K * H) // 32, unroll=8)
            def _cmp(t):
                i = t // (H // 32)
                g = t % (H // 32)
                ew = ebuf[i, pl.ds(g * 16, 16)]
                bc = lambda v: lax.bitcast_convert_type(v, jnp.float32)
                lo = rbuf[i, pl.ds(g * 16, 16)] + bc(ew << 16)
                hi = rbuf[i, pl.ds(H // 2 + g * 16, 16)] + bc(ew & hi_mask)
                mbuf[i, pl.ds(g * 16, 16)] = jnp.maximum(lo, 0.0)
                mbuf[i, pl.ds(H // 2 + g * 16, 16)] = jnp.maximum(hi, 0.0)

        @pl.loop(0, NCHUNK)
        def _chunk(ci):
            pltpu.sync_copy(src_hbm.at[s, ci], srcb)
            pltpu.sync_copy(dst_hbm.at[s, ci], dstb)
            base0 = s * EPT + ci * (CB * K)

            gissue(0, rows_a, gs_a)
            eissue(base0, eb_a, es_a)
            gissue(1, rows_b, gs_b)
            eissue(base0 + K, eb_b, es_b)

            gwait(0, rows_a, gs_a)
            ewait(base0, eb_a, es_a)
            compute(rows_a, eb_a, msg0)
            sissue(0, msg0, ss0)
            gissue(2, rows_a, gs_a)
            eissue(base0 + 2 * K, eb_a, es_a)
            gwait(1, rows_b, gs_b)
            ewait(base0 + K, eb_b, es_b)
            compute(rows_b, eb_b, msg1)
            sissue(1, msg1, ss1)
            gissue(3, rows_b, gs_b)
            eissue(base0 + 3 * K, eb_b, es_b)

            @pl.loop(1, CB // 2)
            def _pair(p):
                b0 = 2 * p
                gwait(b0, rows_a, gs_a)
                ewait(base0 + b0 * K, eb_a, es_a)
                swait(b0 - 2, msg0, ss0)
                compute(rows_a, eb_a, msg0)
                sissue(b0, msg0, ss0)

                @pl.when(p < CB // 2 - 1)
                def _():
                    gissue(b0 + 2, rows_a, gs_a)
                    eissue(base0 + (b0 + 2) * K, eb_a, es_a)

                gwait(b0 + 1, rows_b, gs_b)
                ewait(base0 + (b0 + 1) * K, eb_b, es_b)
                swait(b0 - 1, msg1, ss1)
                compute(rows_b, eb_b, msg1)
                sissue(b0 + 1, msg1, ss1)

                @pl.when(p < CB // 2 - 1)
                def _():
                    gissue(b0 + 3, rows_b, gs_b)
                    eissue(base0 + (b0 + 3) * K, eb_b, es_b)

            swait(CB - 2, msg0, ss0)
            swait(CB - 1, msg1, ss1)

    @pl.when(c == 0)
    def _():
        edge_loop(h0, e0)

    @pl.when(c == 1)
    def _():
        edge_loop(h1, e1)

    plsc.subcore_barrier()

    def writeback(out_ref):
        @pl.loop(s, NZC, step=NS)
        def _wb(i):
            sl = pl.ds(i * ZR, ZR)
            pltpu.sync_copy(agg_sh.at[sl], out_ref.at[sl])

    @pl.when(c == 0)
    def _():
        writeback(out0)

    @pl.when(c == 1)
    def _():
        writeback(out1)


@functools.cache
def _sc_edge_pass():
    return pl.kernel(
        _sc_body,
        out_type=[jax.ShapeDtypeStruct((N, H), jnp.float32)] * 2,
        mesh=plsc.VectorSubcoreMesh(core_axis_name="c", subcore_axis_name="s",
                                    num_cores=2, num_subcores=NS),
        scratch_types=[
            pltpu.VMEM((CB, K), jnp.int32),
            pltpu.VMEM((CB, K), jnp.int32),
            pltpu.VMEM((K, H), jnp.float32),
            pltpu.VMEM((K, H), jnp.float32),
            pltpu.VMEM((K, H // 2), jnp.int32),
            pltpu.VMEM((K, H // 2), jnp.int32),
            pltpu.VMEM((K, H), jnp.float32),
            pltpu.VMEM((K, H), jnp.float32),
            pltpu.VMEM_SHARED((N, H), jnp.float32),
            pltpu.SemaphoreType.DMA,
            pltpu.SemaphoreType.DMA,
            pltpu.SemaphoreType.DMA,
            pltpu.SemaphoreType.DMA,
            pltpu.SemaphoreType.DMA,
            pltpu.SemaphoreType.DMA,
        ],
    )


NB = 2000


def _mlp_body(x_ref, h0_ref, h1_ref, a0_ref, a1_ref,
              w1_ref, b1_ref, w2_ref, b2_ref, o_ref):
    z0 = h0_ref[...] + a0_ref[...]
    z1 = h1_ref[...] + a1_ref[...]
    w1 = w1_ref[...].astype(jnp.bfloat16)
    t = (jnp.dot(z0.astype(jnp.bfloat16), w1[:H, :],
                 preferred_element_type=jnp.float32) +
         jnp.dot(z1.astype(jnp.bfloat16), w1[H:, :],
                 preferred_element_type=jnp.float32) + b1_ref[...])
    a = jnp.maximum(t, 0.0)
    o_ref[...] = x_ref[...] + jnp.dot(
        a.astype(jnp.bfloat16), w2_ref[...].astype(jnp.bfloat16),
        preferred_element_type=jnp.float32) + b2_ref[...]


def _mlp(x, h0, h1, a0, a1, w1, b1, w2, b2):
    half = pl.BlockSpec((NB, H), lambda i: (i, 0))
    full = pl.BlockSpec((NB, D), lambda i: (i, 0))
    wspec = pl.BlockSpec((D, D), lambda i: (0, 0))
    bspec = pl.BlockSpec((1, D), lambda i: (0, 0))
    return pl.pallas_call(
        _mlp_body,
        grid=(N // NB,),
        in_specs=[full, half, half, half, half, wspec, bspec, wspec, bspec],
        out_specs=full,
        out_shape=jax.ShapeDtypeStruct((N, D), jnp.float32),
    )(x, h0, h1, a0, a1, w1, b1, w2, b2)


def kernel(x, edge_index, edge_attr, batch, gn_weight, gn_bias,
           gn_mean_scale, We, be, W1, b1, W2, b2):
    src = edge_index[0].reshape(NS, NCHUNK, CB, K)
    dst = edge_index[1].reshape(NS, NCHUNK, CB, K)
    bat = batch.reshape(N, 1)
    ones = jnp.ones((N, 1), jnp.float32)
    h0, h1, e0, e1 = _pre_sc(x, bat, gn_weight.reshape(1, D),
                             gn_bias.reshape(1, D),
                             gn_mean_scale.reshape(1, D), ones,
                             edge_attr, We, be.reshape(1, D))
    zeros = jnp.zeros((ZR, H), jnp.float32)
    a0, a1 = _sc_edge_pass()(zeros, h0, h1, e0, e1, src, dst)
    permh = jnp.concatenate([jnp.arange(0, H, 2), jnp.arange(1, H, 2)])
    w1m = W1[jnp.concatenate([permh, permh + H]), :]
    return _mlp(x, h0, h1, a0, a1, w1m, b1.reshape(1, D), W2,
                b2.reshape(1, D))


# speedup vs baseline: 2.9380x; 1.0215x over previous
import functools

import jax
import jax.numpy as jnp
from jax import lax
from jax.experimental import pallas as pl
from jax.experimental.pallas import tpu as pltpu
from jax.experimental.pallas import tpu_sc as plsc

N = 10000
E = 160000
D = 256
DE = 16
G = 64
H = D // 2
NS = 16
EPT = E // NS
K = 40
NBLK = EPT // K
CB = 50
NCHUNK = NBLK // CB
ZR = 200
NZC = N // ZR

_PREC = lax.Precision.HIGHEST


def _dotT(a, b):
    return lax.dot_general(a, b, (((0,), (0,)), ((), ())),
                           precision=_PREC, preferred_element_type=jnp.float32)


def _dot(a, b):
    return jnp.dot(a, b, precision=_PREC, preferred_element_type=jnp.float32)


GB = 2000


def _gn_stats_body(x_ref, bat_ref, ones_ref, sum_ref, ssq_ref, cnt_ref):
    i = pl.program_id(0)
    x = x_ref[...]
    bat = bat_ref[...]
    gid = lax.broadcasted_iota(jnp.int32, (1, G), 1)
    p = (bat == gid).astype(jnp.float32)

    @pl.when(i == 0)
    def _():
        sum_ref[...] = jnp.zeros_like(sum_ref)
        ssq_ref[...] = jnp.zeros_like(ssq_ref)
        cnt_ref[...] = jnp.zeros_like(cnt_ref)

    pb = p.astype(jnp.bfloat16)

    def dT(b):
        return lax.dot_general(pb, b.astype(jnp.bfloat16),
                               (((0,), (0,)), ((), ())),
                               preferred_element_type=jnp.float32)

    sum_ref[...] += dT(x)
    ssq_ref[...] += dT(x * x)
    cnt_ref[...] += dT(ones_ref[...])


_HIMASK = -65536


def _gn_norm_body(x_ref, bat_ref, we_ref, wo_ref, be_ref, bo_ref,
                  mse_ref, mso_ref, ms_ref, sele_b_ref, selo_b_ref,
                  sele_f_ref, selo_f_ref,
                  sum_ref, ssq_ref, cnt_ref, h0_ref, h1_ref):
    bat = bat_ref[...]
    gid = lax.broadcasted_iota(jnp.int32, (1, G), 1)
    p = (bat == gid).astype(jnp.float32)
    cnt = jnp.maximum(cnt_ref[...], 1.0)
    mean = sum_ref[...] / cnt
    ms = ms_ref[...]
    var = ssq_ref[...] / cnt + mean * mean * ms * (ms - 2.0)
    rstd = lax.rsqrt(var + 1e-5)
    xb = x_ref[...].astype(jnp.bfloat16)
    xe = jnp.dot(xb, sele_b_ref[...], preferred_element_type=jnp.float32)
    xo = jnp.dot(xb, selo_b_ref[...], preferred_element_type=jnp.float32)
    mean_e = _dot(mean, sele_f_ref[...])
    mean_o = _dot(mean, selo_f_ref[...])
    rstd_e = _dot(rstd, sele_f_ref[...])
    rstd_o = _dot(rstd, selo_f_ref[...])
    out_e = xe - _dot(p, mean_e) * mse_ref[...]
    out_o = xo - _dot(p, mean_o) * mso_ref[...]
    he = we_ref[...] * out_e * _dot(p, rstd_e) + be_ref[...]
    ho = wo_ref[...] * out_o * _dot(p, rstd_o) + bo_ref[...]
    h0_ref[...] = jnp.concatenate([he[:, :H // 2], ho[:, :H // 2]], axis=1)
    h1_ref[...] = jnp.concatenate([he[:, H // 2:], ho[:, H // 2:]], axis=1)


NSTEP_GN = N // GB
NSTEP_EE = 20
EB = E // NSTEP_EE


def _pack_bf16(ve, vo):
    bits_e = lax.bitcast_convert_type(
        ve.astype(jnp.bfloat16).astype(jnp.float32), jnp.int32)
    bits_o = lax.bitcast_convert_type(
        vo.astype(jnp.bfloat16).astype(jnp.float32), jnp.int32)
    return lax.shift_right_logical(bits_e, 16) | (bits_o & _HIMASK)


def _ee_body(ea_ref, wee_ref, weo_ref, bee_ref, beo_ref, e0_ref, e1_ref):
    ea = ea_ref[...].astype(jnp.bfloat16)
    ee = jnp.dot(ea, wee_ref[...], preferred_element_type=jnp.float32)
    eo = jnp.dot(ea, weo_ref[...], preferred_element_type=jnp.float32)
    packed = _pack_bf16(ee + bee_ref[...], eo + beo_ref[...])
    e0_ref[...] = packed[:, :H // 2]
    e1_ref[...] = packed[:, H // 2:]


def _pre_body(x_ref, bat_ref, ones_ref, we_ref, wo_ref, be_ref, bo_ref,
              mse_ref, mso_ref, ms_ref, sele_b_ref, selo_b_ref,
              sele_f_ref, selo_f_ref, ea_ref, wee_ref, weo_ref,
              bee_ref, beo_ref, h0_ref, h1_ref, e0_ref, e1_ref,
              sum_s, ssq_s, cnt_s):
    i = pl.program_id(0)

    @pl.when(i < NSTEP_GN)
    def _():
        _gn_stats_body(x_ref, bat_ref, ones_ref, sum_s, ssq_s, cnt_s)

    @pl.when((i >= NSTEP_GN) & (i < 2 * NSTEP_GN))
    def _():
        _gn_norm_body(x_ref, bat_ref, we_ref, wo_ref, be_ref, bo_ref,
                      mse_ref, mso_ref, ms_ref, sele_b_ref, selo_b_ref,
                      sele_f_ref, selo_f_ref, sum_s, ssq_s, cnt_s,
                      h0_ref, h1_ref)

    @pl.when(i >= 2 * NSTEP_GN)
    def _():
        _ee_body(ea_ref, wee_ref, weo_ref, bee_ref, beo_ref, e0_ref, e1_ref)


def _pre_sc(x, bat, w, b, ms, ones, edge_attr, we, be):
    ev = jnp.arange(0, D, 2)
    od = jnp.arange(1, D, 2)
    sele_f = (jnp.arange(D)[:, None] == ev[None, :]).astype(jnp.float32)
    selo_f = (jnp.arange(D)[:, None] == od[None, :]).astype(jnp.float32)

    def xmap(i):
        return (jnp.clip(jnp.where(i < NSTEP_GN, i, i - NSTEP_GN),
                         0, NSTEP_GN - 1), 0)

    def eamap(i):
        return (jnp.clip(i - 2 * NSTEP_GN, 0, NSTEP_EE - 1), 0)

    def hmap(i):
        return (jnp.clip(i - NSTEP_GN, 0, NSTEP_GN - 1), 0)

    xspec = pl.BlockSpec((GB, D), xmap)
    bspec = pl.BlockSpec((GB, 1), xmap)
    row = pl.BlockSpec((1, D), lambda i: (0, 0))
    hrow = pl.BlockSpec((1, H), lambda i: (0, 0))
    sel = pl.BlockSpec((D, H), lambda i: (0, 0))
    wspec = pl.BlockSpec((DE, H), lambda i: (0, 0))
    return pl.pallas_call(
        _pre_body,
        grid=(2 * NSTEP_GN + NSTEP_EE,),
        in_specs=[xspec, bspec, bspec, hrow, hrow, hrow, hrow, hrow, hrow,
                  row, sel, sel, sel, sel,
                  pl.BlockSpec((EB, DE), eamap), wspec, wspec, hrow, hrow],
        out_specs=[pl.BlockSpec((GB, H), hmap)] * 2 +
                  [pl.BlockSpec((EB, H // 2), eamap)] * 2,
        out_shape=[jax.ShapeDtypeStruct((N, H), jnp.float32)] * 2 +
                  [jax.ShapeDtypeStruct((E, H // 2), jnp.int32)] * 2,
        scratch_shapes=[pltpu.VMEM((G, D), jnp.float32),
                        pltpu.VMEM((G, D), jnp.float32),
                        pltpu.VMEM((G, 1), jnp.float32)],
    )(x, bat, ones, w[:, ev], w[:, od], b[:, ev], b[:, od], ms[:, ev],
      ms[:, od], ms, sele_f.astype(jnp.bfloat16), selo_f.astype(jnp.bfloat16),
      sele_f, selo_f, edge_attr, we[:, ev].astype(jnp.bfloat16),
      we[:, od].astype(jnp.bfloat16), be[:, ev], be[:, od])


def _sc_body(zeros_hbm, h0, h1, e0, e1, src_hbm, dst_hbm, out0, out1,
             srcb, dstb, rows_a, rows_b, eb_a, eb_b, msg0, msg1, agg_sh,
             gs_a, gs_b, es_a, es_b, ss0, ss1):
    c = lax.axis_index("c")
    s = lax.axis_index("s")

    @pl.loop(s, NZC, step=NS)
    def _zero(i):
        pltpu.sync_copy(zeros_hbm, agg_sh.at[pl.ds(i * ZR, ZR)])

    plsc.subcore_barrier()

    def edge_loop(h_ref, e_ref):
        def gissue(cb, rbuf, sem):
            pltpu.async_copy(h_ref.at[srcb.at[cb]], rbuf, sem)

        def gwait(cb, rbuf, sem):
            pltpu.make_async_copy(h_ref.at[srcb.at[cb]], rbuf, sem).wait()

        def eissue(base, ebuf, sem):
            pltpu.async_copy(e_ref.at[pl.ds(base, K)], ebuf, sem)

        def ewait(base, ebuf, sem):
            pltpu.make_async_copy(e_ref.at[pl.ds(base, K)], ebuf, sem).wait()

        def sissue(cb, mbuf, sem):
            pltpu.async_copy(mbuf, agg_sh.at[dstb.at[cb]], sem, add=True)

        def swait(cb, mbuf, sem):
            pltpu.make_async_copy(mbuf, agg_sh.at[dstb.at[cb]], sem).wait()

        def compute(rbuf, ebuf, mbuf):
            hi_mask = -65536

            @pl.loop(0, (K * H) // 32, unroll=8)
            def _cmp(t):
                i = t // (H // 32)
                g = t % (H // 32)
                ew = ebuf[i, pl.ds(g * 16, 16)]
                bc = lambda v: lax.bitcast_convert_type(v, jnp.float32)
                lo = rbuf[i, pl.ds(g * 16, 16)] + bc(ew << 16)
                hi = rbuf[i, pl.ds(H // 2 + g * 16, 16)] + bc(ew & hi_mask)
                mbuf[i, pl.ds(g * 16, 16)] = jnp.maximum(lo, 0.0)
                mbuf[i, pl.ds(H // 2 + g * 16, 16)] = jnp.maximum(hi, 0.0)

        @pl.loop(0, NCHUNK)
        def _chunk(ci):
            pltpu.sync_copy(src_hbm.at[s, ci], srcb)
            pltpu.sync_copy(dst_hbm.at[s, ci], dstb)
            base0 = s * EPT + ci * (CB * K)

            gissue(0, rows_a, gs_a)
            eissue(base0, eb_a, es_a)
            gissue(1, rows_b, gs_b)
            eissue(base0 + K, eb_b, es_b)

            gwait(0, rows_a, gs_a)
            ewait(base0, eb_a, es_a)
            compute(rows_a, eb_a, msg0)
            sissue(0, msg0, ss0)
            gissue(2, rows_a, gs_a)
            eissue(base0 + 2 * K, eb_a, es_a)
            gwait(1, rows_b, gs_b)
            ewait(base0 + K, eb_b, es_b)
            compute(rows_b, eb_b, msg1)
            sissue(1, msg1, ss1)
            gissue(3, rows_b, gs_b)
            eissue(base0 + 3 * K, eb_b, es_b)

            @pl.loop(1, CB // 2)
            def _pair(p):
                b0 = 2 * p
                gwait(b0, rows_a, gs_a)
                ewait(base0 + b0 * K, eb_a, es_a)
                swait(b0 - 2, msg0, ss0)
                compute(rows_a, eb_a, msg0)
                sissue(b0, msg0, ss0)

                @pl.when(p < CB // 2 - 1)
                def _():
                    gissue(b0 + 2, rows_a, gs_a)
                    eissue(base0 + (b0 + 2) * K, eb_a, es_a)

                gwait(b0 + 1, rows_b, gs_b)
                ewait(base0 + (b0 + 1) * K, eb_b, es_b)
                swait(b0 - 1, msg1, ss1)
                compute(rows_b, eb_b, msg1)
                sissue(b0 + 1, msg1, ss1)

                @pl.when(p < CB // 2 - 1)
                def _():
                    gissue(b0 + 3, rows_b, gs_b)
                    eissue(base0 + (b0 + 3) * K, eb_b, es_b)

            swait(CB - 2, msg0, ss0)
            swait(CB - 1, msg1, ss1)

    @pl.when(c == 0)
    def _():
        edge_loop(h0, e0)

    @pl.when(c == 1)
    def _():
        edge_loop(h1, e1)

    plsc.subcore_barrier()

    def writeback(out_ref):
        @pl.loop(s, NZC, step=NS)
        def _wb(i):
            sl = pl.ds(i * ZR, ZR)
            pltpu.sync_copy(agg_sh.at[sl], out_ref.at[sl])

    @pl.when(c == 0)
    def _():
        writeback(out0)

    @pl.when(c == 1)
    def _():
        writeback(out1)


@functools.cache
def _sc_edge_pass():
    return pl.kernel(
        _sc_body,
        out_type=[jax.ShapeDtypeStruct((N, H), jnp.float32)] * 2,
        mesh=plsc.VectorSubcoreMesh(core_axis_name="c", subcore_axis_name="s",
                                    num_cores=2, num_subcores=NS),
        scratch_types=[
            pltpu.VMEM((CB, K), jnp.int32),
            pltpu.VMEM((CB, K), jnp.int32),
            pltpu.VMEM((K, H), jnp.float32),
            pltpu.VMEM((K, H), jnp.float32),
            pltpu.VMEM((K, H // 2), jnp.int32),
            pltpu.VMEM((K, H // 2), jnp.int32),
            pltpu.VMEM((K, H), jnp.float32),
            pltpu.VMEM((K, H), jnp.float32),
            pltpu.VMEM_SHARED((N, H), jnp.float32),
            pltpu.SemaphoreType.DMA,
            pltpu.SemaphoreType.DMA,
            pltpu.SemaphoreType.DMA,
            pltpu.SemaphoreType.DMA,
            pltpu.SemaphoreType.DMA,
            pltpu.SemaphoreType.DMA,
        ],
    )


NB = 2000


def _mlp_body(x_ref, h0_ref, h1_ref, a0_ref, a1_ref,
              w1_ref, b1_ref, w2_ref, b2_ref, o_ref):
    z0 = h0_ref[...] + a0_ref[...]
    z1 = h1_ref[...] + a1_ref[...]
    w1 = w1_ref[...].astype(jnp.bfloat16)
    t = (jnp.dot(z0.astype(jnp.bfloat16), w1[:H, :],
                 preferred_element_type=jnp.float32) +
         jnp.dot(z1.astype(jnp.bfloat16), w1[H:, :],
                 preferred_element_type=jnp.float32) + b1_ref[...])
    a = jnp.maximum(t, 0.0)
    o_ref[...] = x_ref[...] + jnp.dot(
        a.astype(jnp.bfloat16), w2_ref[...].astype(jnp.bfloat16),
        preferred_element_type=jnp.float32) + b2_ref[...]


def _mlp(x, h0, h1, a0, a1, w1, b1, w2, b2):
    half = pl.BlockSpec((NB, H), lambda i: (i, 0))
    full = pl.BlockSpec((NB, D), lambda i: (i, 0))
    wspec = pl.BlockSpec((D, D), lambda i: (0, 0))
    bspec = pl.BlockSpec((1, D), lambda i: (0, 0))
    return pl.pallas_call(
        _mlp_body,
        grid=(N // NB,),
        in_specs=[full, half, half, half, half, wspec, bspec, wspec, bspec],
        out_specs=full,
        out_shape=jax.ShapeDtypeStruct((N, D), jnp.float32),
    )(x, h0, h1, a0, a1, w1, b1, w2, b2)


def kernel(x, edge_index, edge_attr, batch, gn_weight, gn_bias,
           gn_mean_scale, We, be, W1, b1, W2, b2):
    src = edge_index[0].reshape(NS, NCHUNK, CB, K)
    dst = edge_index[1].reshape(NS, NCHUNK, CB, K)
    bat = batch.reshape(N, 1)
    ones = jnp.ones((N, 1), jnp.float32)
    h0, h1, e0, e1 = _pre_sc(x, bat, gn_weight.reshape(1, D),
                             gn_bias.reshape(1, D),
                             gn_mean_scale.reshape(1, D), ones,
                             edge_attr, We, be.reshape(1, D))
    zeros = jnp.zeros((ZR, H), jnp.float32)
    a0, a1 = _sc_edge_pass()(zeros, h0, h1, e0, e1, src, dst)
    permh = jnp.concatenate([jnp.arange(0, H, 2), jnp.arange(1, H, 2)])
    w1m = W1[jnp.concatenate([permh, permh + H]), :]
    return _mlp(x, h0, h1, a0, a1, w1m, b1.reshape(1, D), W2,
                b2.reshape(1, D))


# speedup vs baseline: 3.0031x; 1.0222x over previous
import functools

import jax
import jax.numpy as jnp
from jax import lax
from jax.experimental import pallas as pl
from jax.experimental.pallas import tpu as pltpu
from jax.experimental.pallas import tpu_sc as plsc

N = 10000
E = 160000
D = 256
DE = 16
G = 64
H = D // 2
NS = 16
EPT = E // NS
K = 40
NBLK = EPT // K
CB = 50
NCHUNK = NBLK // CB
ZR = 200
NZC = N // ZR

_PREC = lax.Precision.HIGHEST


def _dotT(a, b):
    return lax.dot_general(a, b, (((0,), (0,)), ((), ())),
                           precision=_PREC, preferred_element_type=jnp.float32)


def _dot(a, b):
    return jnp.dot(a, b, precision=_PREC, preferred_element_type=jnp.float32)


GB = 2000


def _gn_stats_body(x_ref, bat_ref, ones_ref, sum_ref, ssq_ref, cnt_ref):
    i = pl.program_id(0)
    x = x_ref[...]
    bat = bat_ref[...]
    gid = lax.broadcasted_iota(jnp.int32, (1, G), 1)
    p = (bat == gid).astype(jnp.float32)

    @pl.when(i == 0)
    def _():
        sum_ref[...] = jnp.zeros_like(sum_ref)
        ssq_ref[...] = jnp.zeros_like(ssq_ref)
        cnt_ref[...] = jnp.zeros_like(cnt_ref)

    pb = p.astype(jnp.bfloat16)

    def dT(b):
        return lax.dot_general(pb, b.astype(jnp.bfloat16),
                               (((0,), (0,)), ((), ())),
                               preferred_element_type=jnp.float32)

    sum_ref[...] += dT(x)
    ssq_ref[...] += dT(x * x)
    cnt_ref[...] += dT(ones_ref[...])


_HIMASK = -65536


def _gn_norm_body(x_ref, bat_ref, we_ref, wo_ref, be_ref, bo_ref,
                  mse_ref, mso_ref, ms_ref, sele_b_ref, selo_b_ref,
                  sele_f_ref, selo_f_ref,
                  sum_ref, ssq_ref, cnt_ref, h0_ref, h1_ref):
    bat = bat_ref[...]
    gid = lax.broadcasted_iota(jnp.int32, (1, G), 1)
    p = (bat == gid).astype(jnp.float32)
    cnt = jnp.maximum(cnt_ref[...], 1.0)
    mean = sum_ref[...] / cnt
    ms = ms_ref[...]
    var = ssq_ref[...] / cnt + mean * mean * ms * (ms - 2.0)
    rstd = lax.rsqrt(var + 1e-5)
    xb = x_ref[...].astype(jnp.bfloat16)
    xe = jnp.dot(xb, sele_b_ref[...], preferred_element_type=jnp.float32)
    xo = jnp.dot(xb, selo_b_ref[...], preferred_element_type=jnp.float32)
    mean_e = _dot(mean, sele_f_ref[...])
    mean_o = _dot(mean, selo_f_ref[...])
    rstd_e = _dot(rstd, sele_f_ref[...])
    rstd_o = _dot(rstd, selo_f_ref[...])
    pb = p.astype(jnp.bfloat16)

    def bcast(v):
        return jnp.dot(pb, v.astype(jnp.bfloat16),
                       preferred_element_type=jnp.float32)

    out_e = xe - bcast(mean_e) * mse_ref[...]
    out_o = xo - bcast(mean_o) * mso_ref[...]
    he = we_ref[...] * out_e * bcast(rstd_e) + be_ref[...]
    ho = wo_ref[...] * out_o * bcast(rstd_o) + bo_ref[...]
    h0_ref[...] = jnp.concatenate([he[:, :H // 2], ho[:, :H // 2]], axis=1)
    h1_ref[...] = jnp.concatenate([he[:, H // 2:], ho[:, H // 2:]], axis=1)


NSTEP_GN = N // GB
NSTEP_EE = 20
EB = E // NSTEP_EE


def _pack_bf16(ve, vo):
    bits_e = lax.bitcast_convert_type(
        ve.astype(jnp.bfloat16).astype(jnp.float32), jnp.int32)
    bits_o = lax.bitcast_convert_type(
        vo.astype(jnp.bfloat16).astype(jnp.float32), jnp.int32)
    return lax.shift_right_logical(bits_e, 16) | (bits_o & _HIMASK)


def _ee_body(ea_ref, wee_ref, weo_ref, bee_ref, beo_ref, e0_ref, e1_ref):
    ea = ea_ref[...].astype(jnp.bfloat16)
    ee = jnp.dot(ea, wee_ref[...], preferred_element_type=jnp.float32)
    eo = jnp.dot(ea, weo_ref[...], preferred_element_type=jnp.float32)
    packed = _pack_bf16(ee + bee_ref[...], eo + beo_ref[...])
    e0_ref[...] = packed[:, :H // 2]
    e1_ref[...] = packed[:, H // 2:]


def _pre_body(x_ref, bat_ref, ones_ref, we_ref, wo_ref, be_ref, bo_ref,
              mse_ref, mso_ref, ms_ref, sele_b_ref, selo_b_ref,
              sele_f_ref, selo_f_ref, ea_ref, wee_ref, weo_ref,
              bee_ref, beo_ref, h0_ref, h1_ref, e0_ref, e1_ref,
              sum_s, ssq_s, cnt_s):
    i = pl.program_id(0)

    @pl.when(i < NSTEP_GN)
    def _():
        _gn_stats_body(x_ref, bat_ref, ones_ref, sum_s, ssq_s, cnt_s)

    @pl.when((i >= NSTEP_GN) & (i < 2 * NSTEP_GN))
    def _():
        _gn_norm_body(x_ref, bat_ref, we_ref, wo_ref, be_ref, bo_ref,
                      mse_ref, mso_ref, ms_ref, sele_b_ref, selo_b_ref,
                      sele_f_ref, selo_f_ref, sum_s, ssq_s, cnt_s,
                      h0_ref, h1_ref)

    @pl.when(i >= 2 * NSTEP_GN)
    def _():
        _ee_body(ea_ref, wee_ref, weo_ref, bee_ref, beo_ref, e0_ref, e1_ref)


def _pre_sc(x, bat, w, b, ms, ones, edge_attr, we, be):
    ev = jnp.arange(0, D, 2)
    od = jnp.arange(1, D, 2)
    sele_f = (jnp.arange(D)[:, None] == ev[None, :]).astype(jnp.float32)
    selo_f = (jnp.arange(D)[:, None] == od[None, :]).astype(jnp.float32)

    def xmap(i):
        return (jnp.clip(jnp.where(i < NSTEP_GN, i, i - NSTEP_GN),
                         0, NSTEP_GN - 1), 0)

    def eamap(i):
        return (jnp.clip(i - 2 * NSTEP_GN, 0, NSTEP_EE - 1), 0)

    def hmap(i):
        return (jnp.clip(i - NSTEP_GN, 0, NSTEP_GN - 1), 0)

    xspec = pl.BlockSpec((GB, D), xmap)
    bspec = pl.BlockSpec((GB, 1), xmap)
    row = pl.BlockSpec((1, D), lambda i: (0, 0))
    hrow = pl.BlockSpec((1, H), lambda i: (0, 0))
    sel = pl.BlockSpec((D, H), lambda i: (0, 0))
    wspec = pl.BlockSpec((DE, H), lambda i: (0, 0))
    return pl.pallas_call(
        _pre_body,
        grid=(2 * NSTEP_GN + NSTEP_EE,),
        in_specs=[xspec, bspec, bspec, hrow, hrow, hrow, hrow, hrow, hrow,
                  row, sel, sel, sel, sel,
                  pl.BlockSpec((EB, DE), eamap), wspec, wspec, hrow, hrow],
        out_specs=[pl.BlockSpec((GB, H), hmap)] * 2 +
                  [pl.BlockSpec((EB, H // 2), eamap)] * 2,
        out_shape=[jax.ShapeDtypeStruct((N, H), jnp.float32)] * 2 +
                  [jax.ShapeDtypeStruct((E, H // 2), jnp.int32)] * 2,
        scratch_shapes=[pltpu.VMEM((G, D), jnp.float32),
                        pltpu.VMEM((G, D), jnp.float32),
                        pltpu.VMEM((G, 1), jnp.float32)],
    )(x, bat, ones, w[:, ev], w[:, od], b[:, ev], b[:, od], ms[:, ev],
      ms[:, od], ms, sele_f.astype(jnp.bfloat16), selo_f.astype(jnp.bfloat16),
      sele_f, selo_f, edge_attr, we[:, ev].astype(jnp.bfloat16),
      we[:, od].astype(jnp.bfloat16), be[:, ev], be[:, od])


def _sc_body(zeros_hbm, h0, h1, e0, e1, src_hbm, dst_hbm, out0, out1,
             srcb, dstb, rows_a, rows_b, eb_a, eb_b, msg0, msg1, agg_sh,
             gs_a, gs_b, es_a, es_b, ss0, ss1):
    c = lax.axis_index("c")
    s = lax.axis_index("s")

    @pl.loop(s, NZC, step=NS)
    def _zero(i):
        pltpu.sync_copy(zeros_hbm, agg_sh.at[pl.ds(i * ZR, ZR)])

    plsc.subcore_barrier()

    def edge_loop(h_ref, e_ref):
        def gissue(cb, rbuf, sem):
            pltpu.async_copy(h_ref.at[srcb.at[cb]], rbuf, sem)

        def gwait(cb, rbuf, sem):
            pltpu.make_async_copy(h_ref.at[srcb.at[cb]], rbuf, sem).wait()

        def eissue(base, ebuf, sem):
            pltpu.async_copy(e_ref.at[pl.ds(base, K)], ebuf, sem)

        def ewait(base, ebuf, sem):
            pltpu.make_async_copy(e_ref.at[pl.ds(base, K)], ebuf, sem).wait()

        def sissue(cb, mbuf, sem):
            pltpu.async_copy(mbuf, agg_sh.at[dstb.at[cb]], sem, add=True)

        def swait(cb, mbuf, sem):
            pltpu.make_async_copy(mbuf, agg_sh.at[dstb.at[cb]], sem).wait()

        def compute(rbuf, ebuf, mbuf):
            hi_mask = -65536

            @pl.loop(0, (K * H) // 32, unroll=8)
            def _cmp(t):
                i = t // (H // 32)
                g = t % (H // 32)
                ew = ebuf[i, pl.ds(g * 16, 16)]
                bc = lambda v: lax.bitcast_convert_type(v, jnp.float32)
                lo = rbuf[i, pl.ds(g * 16, 16)] + bc(ew << 16)
                hi = rbuf[i, pl.ds(H // 2 + g * 16, 16)] + bc(ew & hi_mask)
                mbuf[i, pl.ds(g * 16, 16)] = jnp.maximum(lo, 0.0)
                mbuf[i, pl.ds(H // 2 + g * 16, 16)] = jnp.maximum(hi, 0.0)

        @pl.loop(0, NCHUNK)
        def _chunk(ci):
            pltpu.sync_copy(src_hbm.at[s, ci], srcb)
            pltpu.sync_copy(dst_hbm.at[s, ci], dstb)
            base0 = s * EPT + ci * (CB * K)

            gissue(0, rows_a, gs_a)
            eissue(base0, eb_a, es_a)
            gissue(1, rows_b, gs_b)
            eissue(base0 + K, eb_b, es_b)

            gwait(0, rows_a, gs_a)
            ewait(base0, eb_a, es_a)
            compute(rows_a, eb_a, msg0)
            sissue(0, msg0, ss0)
            gissue(2, rows_a, gs_a)
            eissue(base0 + 2 * K, eb_a, es_a)
            gwait(1, rows_b, gs_b)
            ewait(base0 + K, eb_b, es_b)
            compute(rows_b, eb_b, msg1)
            sissue(1, msg1, ss1)
            gissue(3, rows_b, gs_b)
            eissue(base0 + 3 * K, eb_b, es_b)

            @pl.loop(1, CB // 2)
            def _pair(p):
                b0 = 2 * p
                gwait(b0, rows_a, gs_a)
                ewait(base0 + b0 * K, eb_a, es_a)
                swait(b0 - 2, msg0, ss0)
                compute(rows_a, eb_a, msg0)
                sissue(b0, msg0, ss0)

                @pl.when(p < CB // 2 - 1)
                def _():
                    gissue(b0 + 2, rows_a, gs_a)
                    eissue(base0 + (b0 + 2) * K, eb_a, es_a)

                gwait(b0 + 1, rows_b, gs_b)
                ewait(base0 + (b0 + 1) * K, eb_b, es_b)
                swait(b0 - 1, msg1, ss1)
                compute(rows_b, eb_b, msg1)
                sissue(b0 + 1, msg1, ss1)

                @pl.when(p < CB // 2 - 1)
                def _():
                    gissue(b0 + 3, rows_b, gs_b)
                    eissue(base0 + (b0 + 3) * K, eb_b, es_b)

            swait(CB - 2, msg0, ss0)
            swait(CB - 1, msg1, ss1)

    @pl.when(c == 0)
    def _():
        edge_loop(h0, e0)

    @pl.when(c == 1)
    def _():
        edge_loop(h1, e1)

    plsc.subcore_barrier()

    def writeback(out_ref):
        @pl.loop(s, NZC, step=NS)
        def _wb(i):
            sl = pl.ds(i * ZR, ZR)
            pltpu.sync_copy(agg_sh.at[sl], out_ref.at[sl])

    @pl.when(c == 0)
    def _():
        writeback(out0)

    @pl.when(c == 1)
    def _():
        writeback(out1)


@functools.cache
def _sc_edge_pass():
    return pl.kernel(
        _sc_body,
        out_type=[jax.ShapeDtypeStruct((N, H), jnp.float32)] * 2,
        mesh=plsc.VectorSubcoreMesh(core_axis_name="c", subcore_axis_name="s",
                                    num_cores=2, num_subcores=NS),
        scratch_types=[
            pltpu.VMEM((CB, K), jnp.int32),
            pltpu.VMEM((CB, K), jnp.int32),
            pltpu.VMEM((K, H), jnp.float32),
            pltpu.VMEM((K, H), jnp.float32),
            pltpu.VMEM((K, H // 2), jnp.int32),
            pltpu.VMEM((K, H // 2), jnp.int32),
            pltpu.VMEM((K, H), jnp.float32),
            pltpu.VMEM((K, H), jnp.float32),
            pltpu.VMEM_SHARED((N, H), jnp.float32),
            pltpu.SemaphoreType.DMA,
            pltpu.SemaphoreType.DMA,
            pltpu.SemaphoreType.DMA,
            pltpu.SemaphoreType.DMA,
            pltpu.SemaphoreType.DMA,
            pltpu.SemaphoreType.DMA,
        ],
    )


NB = 2000


def _mlp_body(x_ref, h0_ref, h1_ref, a0_ref, a1_ref,
              w1_ref, b1_ref, w2_ref, b2_ref, o_ref):
    z0 = h0_ref[...] + a0_ref[...]
    z1 = h1_ref[...] + a1_ref[...]
    w1 = w1_ref[...].astype(jnp.bfloat16)
    t = (jnp.dot(z0.astype(jnp.bfloat16), w1[:H, :],
                 preferred_element_type=jnp.float32) +
         jnp.dot(z1.astype(jnp.bfloat16), w1[H:, :],
                 preferred_element_type=jnp.float32) + b1_ref[...])
    a = jnp.maximum(t, 0.0)
    o_ref[...] = x_ref[...] + jnp.dot(
        a.astype(jnp.bfloat16), w2_ref[...].astype(jnp.bfloat16),
        preferred_element_type=jnp.float32) + b2_ref[...]


def _mlp(x, h0, h1, a0, a1, w1, b1, w2, b2):
    half = pl.BlockSpec((NB, H), lambda i: (i, 0))
    full = pl.BlockSpec((NB, D), lambda i: (i, 0))
    wspec = pl.BlockSpec((D, D), lambda i: (0, 0))
    bspec = pl.BlockSpec((1, D), lambda i: (0, 0))
    return pl.pallas_call(
        _mlp_body,
        grid=(N // NB,),
        in_specs=[full, half, half, half, half, wspec, bspec, wspec, bspec],
        out_specs=full,
        out_shape=jax.ShapeDtypeStruct((N, D), jnp.float32),
    )(x, h0, h1, a0, a1, w1, b1, w2, b2)


def kernel(x, edge_index, edge_attr, batch, gn_weight, gn_bias,
           gn_mean_scale, We, be, W1, b1, W2, b2):
    src = edge_index[0].reshape(NS, NCHUNK, CB, K)
    dst = edge_index[1].reshape(NS, NCHUNK, CB, K)
    bat = batch.reshape(N, 1)
    ones = jnp.ones((N, 1), jnp.float32)
    h0, h1, e0, e1 = _pre_sc(x, bat, gn_weight.reshape(1, D),
                             gn_bias.reshape(1, D),
                             gn_mean_scale.reshape(1, D), ones,
                             edge_attr, We, be.reshape(1, D))
    zeros = jnp.zeros((ZR, H), jnp.float32)
    a0, a1 = _sc_edge_pass()(zeros, h0, h1, e0, e1, src, dst)
    permh = jnp.concatenate([jnp.arange(0, H, 2), jnp.arange(1, H, 2)])
    w1m = W1[jnp.concatenate([permh, permh + H]), :]
    return _mlp(x, h0, h1, a0, a1, w1m, b1.reshape(1, D), W2,
                b2.reshape(1, D))


# speedup vs baseline: 3.0035x; 1.0001x over previous
import functools

import jax
import jax.numpy as jnp
from jax import lax
from jax.experimental import pallas as pl
from jax.experimental.pallas import tpu as pltpu
from jax.experimental.pallas import tpu_sc as plsc

N = 10000
E = 160000
D = 256
DE = 16
G = 64
H = D // 2
NS = 16
EPT = E // NS
K = 40
NBLK = EPT // K
CB = 50
NCHUNK = NBLK // CB
ZR = 200
NZC = N // ZR

_PREC = lax.Precision.HIGHEST


def _dot(a, b):
    return jnp.dot(a, b, precision=_PREC, preferred_element_type=jnp.float32)


GB = 2000


def _gn_stats_body(x_ref, bat_ref, ones_ref, sum_ref, ssq_ref, cnt_ref):
    i = pl.program_id(0)
    x = x_ref[...]
    bat = bat_ref[...]
    gid = lax.broadcasted_iota(jnp.int32, (1, G), 1)
    p = (bat == gid).astype(jnp.float32)

    @pl.when(i == 0)
    def _():
        sum_ref[...] = jnp.zeros_like(sum_ref)
        ssq_ref[...] = jnp.zeros_like(ssq_ref)
        cnt_ref[...] = jnp.zeros_like(cnt_ref)

    pb = p.astype(jnp.bfloat16)

    def dT(b):
        return lax.dot_general(pb, b.astype(jnp.bfloat16),
                               (((0,), (0,)), ((), ())),
                               preferred_element_type=jnp.float32)

    sum_ref[...] += dT(x)
    ssq_ref[...] += dT(x * x)
    cnt_ref[...] += dT(ones_ref[...])


_HIMASK = -65536


def _gn_norm_body(x_ref, bat_ref, we_ref, wo_ref, be_ref, bo_ref,
                  mse_ref, mso_ref, ms_ref, sele_b_ref, selo_b_ref,
                  sele_f_ref, selo_f_ref,
                  sum_ref, ssq_ref, cnt_ref, h0_ref, h1_ref):
    bat = bat_ref[...]
    gid = lax.broadcasted_iota(jnp.int32, (1, G), 1)
    p = (bat == gid).astype(jnp.float32)
    cnt = jnp.maximum(cnt_ref[...], 1.0)
    mean = sum_ref[...] / cnt
    ms = ms_ref[...]
    var = ssq_ref[...] / cnt + mean * mean * ms * (ms - 2.0)
    rstd = lax.rsqrt(var + 1e-5)
    xb = x_ref[...].astype(jnp.bfloat16)
    xe = jnp.dot(xb, sele_b_ref[...], preferred_element_type=jnp.float32)
    xo = jnp.dot(xb, selo_b_ref[...], preferred_element_type=jnp.float32)
    mean_e = _dot(mean, sele_f_ref[...])
    mean_o = _dot(mean, selo_f_ref[...])
    rstd_e = _dot(rstd, sele_f_ref[...])
    rstd_o = _dot(rstd, selo_f_ref[...])
    pb = p.astype(jnp.bfloat16)

    def bcast(v):
        return jnp.dot(pb, v.astype(jnp.bfloat16),
                       preferred_element_type=jnp.float32)

    out_e = xe - bcast(mean_e) * mse_ref[...]
    out_o = xo - bcast(mean_o) * mso_ref[...]
    he = we_ref[...] * out_e * bcast(rstd_e) + be_ref[...]
    ho = wo_ref[...] * out_o * bcast(rstd_o) + bo_ref[...]
    h0_ref[...] = jnp.concatenate([he[:, :H // 2], ho[:, :H // 2]], axis=1)
    h1_ref[...] = jnp.concatenate([he[:, H // 2:], ho[:, H // 2:]], axis=1)


NSTEP_GN = N // GB
NSTEP_EE = 20
EB = E // NSTEP_EE


def _pack_bf16(ve, vo):
    bits_e = lax.bitcast_convert_type(
        ve.astype(jnp.bfloat16).astype(jnp.float32), jnp.int32)
    bits_o = lax.bitcast_convert_type(
        vo.astype(jnp.bfloat16).astype(jnp.float32), jnp.int32)
    return lax.shift_right_logical(bits_e, 16) | (bits_o & _HIMASK)


def _ee_body(ea_ref, wee_ref, weo_ref, bee_ref, beo_ref, e0_ref, e1_ref):
    ea = ea_ref[...].astype(jnp.bfloat16)
    ee = jnp.dot(ea, wee_ref[...], preferred_element_type=jnp.float32)
    eo = jnp.dot(ea, weo_ref[...], preferred_element_type=jnp.float32)
    packed = _pack_bf16(ee + bee_ref[...], eo + beo_ref[...])
    e0_ref[...] = packed[:, :H // 2]
    e1_ref[...] = packed[:, H // 2:]


def _pre_body(x_ref, bat_ref, ones_ref, we_ref, wo_ref, be_ref, bo_ref,
              mse_ref, mso_ref, ms_ref, sele_b_ref, selo_b_ref,
              sele_f_ref, selo_f_ref, ea_ref, wee_ref, weo_ref,
              bee_ref, beo_ref, h0_ref, h1_ref, e0_ref, e1_ref,
              sum_s, ssq_s, cnt_s):
    i = pl.program_id(0)

    @pl.when(i < NSTEP_GN)
    def _():
        _gn_stats_body(x_ref, bat_ref, ones_ref, sum_s, ssq_s, cnt_s)

    @pl.when((i >= NSTEP_GN) & (i < 2 * NSTEP_GN))
    def _():
        _gn_norm_body(x_ref, bat_ref, we_ref, wo_ref, be_ref, bo_ref,
                      mse_ref, mso_ref, ms_ref, sele_b_ref, selo_b_ref,
                      sele_f_ref, selo_f_ref, sum_s, ssq_s, cnt_s,
                      h0_ref, h1_ref)

    @pl.when(i >= 2 * NSTEP_GN)
    def _():
        _ee_body(ea_ref, wee_ref, weo_ref, bee_ref, beo_ref, e0_ref, e1_ref)


def _pre_sc(x, bat, w, b, ms, ones, edge_attr, we, be):
    ev = jnp.arange(0, D, 2)
    od = jnp.arange(1, D, 2)
    sele_f = (jnp.arange(D)[:, None] == ev[None, :]).astype(jnp.float32)
    selo_f = (jnp.arange(D)[:, None] == od[None, :]).astype(jnp.float32)

    def xmap(i):
        return (jnp.clip(jnp.where(i < NSTEP_GN, i, i - NSTEP_GN),
                         0, NSTEP_GN - 1), 0)

    def eamap(i):
        return (jnp.clip(i - 2 * NSTEP_GN, 0, NSTEP_EE - 1), 0)

    def hmap(i):
        return (jnp.clip(i - NSTEP_GN, 0, NSTEP_GN - 1), 0)

    xspec = pl.BlockSpec((GB, D), xmap)
    bspec = pl.BlockSpec((GB, 1), xmap)
    row = pl.BlockSpec((1, D), lambda i: (0, 0))
    hrow = pl.BlockSpec((1, H), lambda i: (0, 0))
    sel = pl.BlockSpec((D, H), lambda i: (0, 0))
    wspec = pl.BlockSpec((DE, H), lambda i: (0, 0))
    return pl.pallas_call(
        _pre_body,
        grid=(2 * NSTEP_GN + NSTEP_EE,),
        in_specs=[xspec, bspec, bspec, hrow, hrow, hrow, hrow, hrow, hrow,
                  row, sel, sel, sel, sel,
                  pl.BlockSpec((EB, DE), eamap), wspec, wspec, hrow, hrow],
        out_specs=[pl.BlockSpec((GB, H), hmap)] * 2 +
                  [pl.BlockSpec((EB, H // 2), eamap)] * 2,
        out_shape=[jax.ShapeDtypeStruct((N, H), jnp.float32)] * 2 +
                  [jax.ShapeDtypeStruct((E, H // 2), jnp.int32)] * 2,
        scratch_shapes=[pltpu.VMEM((G, D), jnp.float32),
                        pltpu.VMEM((G, D), jnp.float32),
                        pltpu.VMEM((G, 1), jnp.float32)],
    )(x, bat, ones, w[:, ev], w[:, od], b[:, ev], b[:, od], ms[:, ev],
      ms[:, od], ms, sele_f.astype(jnp.bfloat16), selo_f.astype(jnp.bfloat16),
      sele_f, selo_f, edge_attr, we[:, ev].astype(jnp.bfloat16),
      we[:, od].astype(jnp.bfloat16), be[:, ev], be[:, od])


def _sc_body(zeros_hbm, h0, h1, e0, e1, src_hbm, dst_hbm, out0, out1,
             srcb, dstb, rows_a, rows_b, eb_a, eb_b, msg0, msg1, agg_sh,
             gs_a, gs_b, es_a, es_b, ss0, ss1):
    c = lax.axis_index("c")
    s = lax.axis_index("s")

    @pl.loop(s, NZC, step=NS)
    def _zero(i):
        pltpu.sync_copy(zeros_hbm, agg_sh.at[pl.ds(i * ZR, ZR)])

    plsc.subcore_barrier()

    def edge_loop(h_ref, e_ref):
        def gissue(cb, rbuf, sem):
            pltpu.async_copy(h_ref.at[srcb.at[cb]], rbuf, sem)

        def gwait(cb, rbuf, sem):
            pltpu.make_async_copy(h_ref.at[srcb.at[cb]], rbuf, sem).wait()

        def eissue(base, ebuf, sem):
            pltpu.async_copy(e_ref.at[pl.ds(base, K)], ebuf, sem)

        def ewait(base, ebuf, sem):
            pltpu.make_async_copy(e_ref.at[pl.ds(base, K)], ebuf, sem).wait()

        def sissue(cb, mbuf, sem):
            pltpu.async_copy(mbuf, agg_sh.at[dstb.at[cb]], sem, add=True)

        def swait(cb, mbuf, sem):
            pltpu.make_async_copy(mbuf, agg_sh.at[dstb.at[cb]], sem).wait()

        def compute(rbuf, ebuf, mbuf):
            hi_mask = -65536

            @pl.loop(0, (K * H) // 32, unroll=8)
            def _cmp(t):
                i = t // (H // 32)
                g = t % (H // 32)
                ew = ebuf[i, pl.ds(g * 16, 16)]
                bc = lambda v: lax.bitcast_convert_type(v, jnp.float32)
                lo = rbuf[i, pl.ds(g * 16, 16)] + bc(ew << 16)
                hi = rbuf[i, pl.ds(H // 2 + g * 16, 16)] + bc(ew & hi_mask)
                mbuf[i, pl.ds(g * 16, 16)] = jnp.maximum(lo, 0.0)
                mbuf[i, pl.ds(H // 2 + g * 16, 16)] = jnp.maximum(hi, 0.0)

        @pl.loop(0, NCHUNK)
        def _chunk(ci):
            pltpu.sync_copy(src_hbm.at[s, ci], srcb)
            pltpu.sync_copy(dst_hbm.at[s, ci], dstb)
            base0 = s * EPT + ci * (CB * K)

            gissue(0, rows_a, gs_a)
            eissue(base0, eb_a, es_a)
            gissue(1, rows_b, gs_b)
            eissue(base0 + K, eb_b, es_b)

            gwait(0, rows_a, gs_a)
            ewait(base0, eb_a, es_a)
            compute(rows_a, eb_a, msg0)
            sissue(0, msg0, ss0)
            gissue(2, rows_a, gs_a)
            eissue(base0 + 2 * K, eb_a, es_a)
            gwait(1, rows_b, gs_b)
            ewait(base0 + K, eb_b, es_b)
            compute(rows_b, eb_b, msg1)
            sissue(1, msg1, ss1)
            gissue(3, rows_b, gs_b)
            eissue(base0 + 3 * K, eb_b, es_b)

            @pl.loop(1, CB // 2)
            def _pair(p):
                b0 = 2 * p
                gwait(b0, rows_a, gs_a)
                ewait(base0 + b0 * K, eb_a, es_a)
                swait(b0 - 2, msg0, ss0)
                compute(rows_a, eb_a, msg0)
                sissue(b0, msg0, ss0)

                @pl.when(p < CB // 2 - 1)
                def _():
                    gissue(b0 + 2, rows_a, gs_a)
                    eissue(base0 + (b0 + 2) * K, eb_a, es_a)

                gwait(b0 + 1, rows_b, gs_b)
                ewait(base0 + (b0 + 1) * K, eb_b, es_b)
                swait(b0 - 1, msg1, ss1)
                compute(rows_b, eb_b, msg1)
                sissue(b0 + 1, msg1, ss1)

                @pl.when(p < CB // 2 - 1)
                def _():
                    gissue(b0 + 3, rows_b, gs_b)
                    eissue(base0 + (b0 + 3) * K, eb_b, es_b)

            swait(CB - 2, msg0, ss0)
            swait(CB - 1, msg1, ss1)

    @pl.when(c == 0)
    def _():
        edge_loop(h0, e0)

    @pl.when(c == 1)
    def _():
        edge_loop(h1, e1)

    plsc.subcore_barrier()

    def writeback(out_ref):
        @pl.loop(s, NZC, step=NS)
        def _wb(i):
            sl = pl.ds(i * ZR, ZR)
            pltpu.sync_copy(agg_sh.at[sl], out_ref.at[sl])

    @pl.when(c == 0)
    def _():
        writeback(out0)

    @pl.when(c == 1)
    def _():
        writeback(out1)


@functools.cache
def _sc_edge_pass():
    return pl.kernel(
        _sc_body,
        out_type=[jax.ShapeDtypeStruct((N, H), jnp.float32)] * 2,
        mesh=plsc.VectorSubcoreMesh(core_axis_name="c", subcore_axis_name="s",
                                    num_cores=2, num_subcores=NS),
        scratch_types=[
            pltpu.VMEM((CB, K), jnp.int32),
            pltpu.VMEM((CB, K), jnp.int32),
            pltpu.VMEM((K, H), jnp.float32),
            pltpu.VMEM((K, H), jnp.float32),
            pltpu.VMEM((K, H // 2), jnp.int32),
            pltpu.VMEM((K, H // 2), jnp.int32),
            pltpu.VMEM((K, H), jnp.float32),
            pltpu.VMEM((K, H), jnp.float32),
            pltpu.VMEM_SHARED((N, H), jnp.float32),
            pltpu.SemaphoreType.DMA,
            pltpu.SemaphoreType.DMA,
            pltpu.SemaphoreType.DMA,
            pltpu.SemaphoreType.DMA,
            pltpu.SemaphoreType.DMA,
            pltpu.SemaphoreType.DMA,
        ],
    )


NB = 2000


def _mlp_body(x_ref, h0_ref, h1_ref, a0_ref, a1_ref,
              w1_ref, b1_ref, w2_ref, b2_ref, o_ref):
    z0 = h0_ref[...] + a0_ref[...]
    z1 = h1_ref[...] + a1_ref[...]
    w1 = w1_ref[...].astype(jnp.bfloat16)
    t = (jnp.dot(z0.astype(jnp.bfloat16), w1[:H, :],
                 preferred_element_type=jnp.float32) +
         jnp.dot(z1.astype(jnp.bfloat16), w1[H:, :],
                 preferred_element_type=jnp.float32) + b1_ref[...])
    a = jnp.maximum(t, 0.0)
    o_ref[...] = x_ref[...] + jnp.dot(
        a.astype(jnp.bfloat16), w2_ref[...].astype(jnp.bfloat16),
        preferred_element_type=jnp.float32) + b2_ref[...]


def _mlp(x, h0, h1, a0, a1, w1, b1, w2, b2):
    half = pl.BlockSpec((NB, H), lambda i: (i, 0))
    full = pl.BlockSpec((NB, D), lambda i: (i, 0))
    wspec = pl.BlockSpec((D, D), lambda i: (0, 0))
    bspec = pl.BlockSpec((1, D), lambda i: (0, 0))
    return pl.pallas_call(
        _mlp_body,
        grid=(N // NB,),
        in_specs=[full, half, half, half, half, wspec, bspec, wspec, bspec],
        out_specs=full,
        out_shape=jax.ShapeDtypeStruct((N, D), jnp.float32),
    )(x, h0, h1, a0, a1, w1, b1, w2, b2)


def kernel(x, edge_index, edge_attr, batch, gn_weight, gn_bias,
           gn_mean_scale, We, be, W1, b1, W2, b2):
    src = edge_index[0].reshape(NS, NCHUNK, CB, K)
    dst = edge_index[1].reshape(NS, NCHUNK, CB, K)
    bat = batch.reshape(N, 1)
    ones = jnp.ones((N, 1), jnp.float32)
    h0, h1, e0, e1 = _pre_sc(x, bat, gn_weight.reshape(1, D),
                             gn_bias.reshape(1, D),
                             gn_mean_scale.reshape(1, D), ones,
                             edge_attr, We, be.reshape(1, D))
    zeros = jnp.zeros((ZR, H), jnp.float32)
    a0, a1 = _sc_edge_pass()(zeros, h0, h1, e0, e1, src, dst)
    permh = jnp.concatenate([jnp.arange(0, H, 2), jnp.arange(1, H, 2)])
    w1m = W1[jnp.concatenate([permh, permh + H]), :]
    return _mlp(x, h0, h1, a0, a1, w1m, b1.reshape(1, D), W2,
                b2.reshape(1, D))
